# baseline (speedup 1.0000x reference)
import functools

import jax
import jax.numpy as jnp
from jax import lax
from jax.experimental import pallas as pl
from jax.experimental.pallas import tpu as pltpu
from jax.experimental.pallas import tpu_sc as plsc

XD = 128
HD = 32
ZD = 16
T = 2
N = 10000
E = 320000
NP = 10240
NB = 2048
TB = NP // NB
BM = 2048
DB = NP // BM
NC = 2
NS = 16
NW = NC * NS
EPW = E // NW
CH = 80
NCH = EPW // CH
KB = 5
STRIPE = NP // NS
SEPS = 1e-8


def _softplus(v):
    return jnp.maximum(v, 0.0) + jnp.log1p(jnp.exp(-jnp.abs(v)))


def _softplus_sum(v):
    u = jnp.exp(-jnp.abs(v).astype(jnp.bfloat16))
    w = jnp.log(1.0 + u)
    return jnp.maximum(v, 0.0) + w.astype(jnp.float32)


def _make_sc_pass(fs, nt, with_gather):
    mesh = plsc.VectorSubcoreMesh(core_axis_name="c", subcore_axis_name="s",
                                  num_cores=NC, num_subcores=NS)
    out_type = tuple(
        jax.ShapeDtypeStruct((NC, nt, NP, f), jnp.float32) for f in fs)
    kb = KB
    scratch = [
        pltpu.VMEM((NCH, CH), jnp.int32),
        pltpu.VMEM((NCH, CH), jnp.int32),
    ]
    nbuf = kb if with_gather else 1
    for f in fs:
        for _ in range(nbuf):
            scratch.append(pltpu.VMEM((CH, f), jnp.float32))
            scratch.append(pltpu.SemaphoreType.DMA)
        for _ in range(kb):
            scratch.append(pltpu.SemaphoreType.DMA)
        scratch.append(pltpu.VMEM_SHARED((NP, f), jnp.float32))

    @functools.partial(
        pl.kernel, out_type=out_type, mesh=mesh, scratch_types=scratch,
        compiler_params=pltpu.CompilerParams(use_tc_tiling_on_sc=False))
    def k(*refs):
        nu = len(fs)
        ng = nu * nt if with_gather else 0
        us = refs[:ng]
        gidx_hbm = refs[ng]
        sidx_hbm = refs[ng + 1]
        outs = refs[ng + 2:ng + 2 + nu]
        sc = refs[ng + 2 + nu:]
        gi_v, si_v = sc[0], sc[1]
        per_u = 2 * nbuf + kb + 1
        rows = []
        sems = []
        ssems = []
        accs = []
        for i in range(nu):
            grp = sc[2 + i * per_u:2 + (i + 1) * per_u]
            rows.append([grp[2 * k] for k in range(nbuf)])
            sems.append([grp[2 * k + 1] for k in range(nbuf)])
            ssems.append([grp[2 * nbuf + k] for k in range(kb)])
            accs.append(grp[2 * nbuf + kb])

        cid = lax.axis_index("c")
        sid = lax.axis_index("s")
        wid = sid * NC + cid

        def _fill_rows(val):
            for i, f in enumerate(fs):
                def frow(j, _, _r=rows[i][0], _f=f, _v=val):
                    for kk in range(_f // 16):
                        _r[j, pl.ds(16 * kk, 16)] = jnp.full(
                            (16,), _v, jnp.float32)
                    return 0
                lax.fori_loop(0, CH, frow, 0)

        def _start_gather(i, t, k, j):
            pltpu.async_copy(us[i * nt + t].at[gi_v.at[j]], rows[i][k],
                             sems[i][k])

        def _wait_gather(i, t, k):
            pltpu.make_async_copy(us[i * nt + t].at[pl.ds(0, CH)],
                                  rows[i][k], sems[i][k]).wait()

        def _start_scatter(i, b, k, j):
            pltpu.async_copy(rows[i][b], accs[i].at[si_v.at[j]],
                             ssems[i][k], add=True)

        def _wait_scatter(i, b, k):
            pltpu.make_async_copy(rows[i][b], accs[i].at[pl.ds(0, CH)],
                                  ssems[i][k]).wait()

        for t in range(nt):
            _fill_rows(0.0)
            for i in range(nu):
                for kk in range(STRIPE // CH):
                    pltpu.sync_copy(
                        rows[i][0],
                        accs[i].at[pl.ds(sid * STRIPE + kk * CH, CH)])
            if not with_gather:
                _fill_rows(1.0)
            plsc.subcore_barrier()
            pltpu.sync_copy(gidx_hbm.at[t, wid], gi_v)
            pltpu.sync_copy(sidx_hbm.at[t, wid], si_v)

            if with_gather:
                for k in range(kb):
                    for i in range(nu):
                        _start_gather(i, t, k, k)

                def group(g, _, _t=t):
                    for k in range(kb):
                        j = g * kb + k
                        for i in range(nu):
                            _wait_gather(i, _t, k)
                            _start_scatter(i, k, k, j)
                        pk = (k - 1) % kb
                        pj = j - 1 + kb
                        cond = (pj < NCH) if k >= 1 else (
                            (g >= 1) & (pj < NCH))

                        @pl.when(cond)
                        def _(_pk=pk, _pj=pj, _tt=_t):
                            for i in range(nu):
                                _wait_scatter(i, _pk, _pk)
                                _start_gather(i, _tt, _pk, _pj)
                    return 0
                lax.fori_loop(0, NCH // kb, group, 0)
                for k in range(kb):
                    for i in range(nu):
                        _wait_scatter(i, k, k)
            else:
                def group0(g, _):
                    for k in range(kb):
                        j = g * kb + k

                        @pl.when(g >= 1)
                        def _(_k=k):
                            for i in range(nu):
                                _wait_scatter(i, 0, _k)
                        for i in range(nu):
                            _start_scatter(i, 0, k, j)
                    return 0
                lax.fori_loop(0, NCH // kb, group0, 0)
                for k in range(kb):
                    for i in range(nu):
                        _wait_scatter(i, 0, k)
            plsc.subcore_barrier()
            for i in range(nu):
                pltpu.sync_copy(
                    accs[i].at[pl.ds(sid * STRIPE, STRIPE)],
                    outs[i].at[cid, t, pl.ds(sid * STRIPE, STRIPE)])
            plsc.subcore_barrier()

    return k


@functools.lru_cache(maxsize=None)
def _get_sc_pass(fs_key, nt, with_gather):
    return _make_sc_pass(list(fs_key), nt, with_gather)


def _tc1_body(x_ref, degp_ref, w_ref, b_ref, encw_ref,
              phi_ref, ub_ref, dinv_ref):
    deg = degp_ref[0, 0] + degp_ref[1, 0] + 1.0
    dinv = lax.rsqrt(deg)
    dinv_ref[0] = dinv
    phi = jnp.maximum(
        jnp.dot(x_ref[0], w_ref[...],
                preferred_element_type=jnp.float32) + b_ref[...], 0.0)
    phi_ref[0] = phi
    ub_ref[0] = dinv[:, :1] * jnp.dot(phi, encw_ref[...],
                                      preferred_element_type=jnp.float32)


def _tc2_body(t, accb_ref, ub_ref, dinv_ref, wc_ref, uc_ref):
    d1 = dinv_ref[0][:, :1]
    enc = jnp.maximum(d1 * (accb_ref[0, 0] + accb_ref[1, 0] + ub_ref[0]), 0.0)
    uc_ref[...] = d1 * jnp.dot(enc, wc_ref[...],
                               preferred_element_type=jnp.float32)


def _tc3_body(t, accc_ref, uc_ref, dinv_ref, eps_ref, msk_ref, phi_ref,
              wz_ref, bz_ref, wda_ref, wdb_ref, pmu_ref, pstd_ref,
              mu_ref, zd_ref, udg_ref, kld_ref):
    i = pl.program_id(0)
    d1 = dinv_ref[0][:, :1]
    musd = d1 * (accc_ref[0, 0] + accc_ref[1, 0] + uc_ref[...])
    mu = musd[:, :ZD]
    std = _softplus(musd[:, ZD:])
    mu_ref[...] = mu
    z = mu + eps_ref[0] * std
    phiz = jnp.maximum(
        jnp.dot(z, wz_ref[...], preferred_element_type=jnp.float32)
        + bz_ref[...], 0.0)
    zd = msk_ref[0] * (2.0 * z)
    zd_ref[...] = zd
    udg_ref[...] = d1 * (
        jnp.dot(phi_ref[0], wda_ref[0], preferred_element_type=jnp.float32)
        + jnp.dot(phiz, wdb_ref[0], preferred_element_type=jnp.float32))
    pmu = pmu_ref[...]
    pstd = pstd_ref[...]
    term = (2.0 * (jnp.log(pstd + SEPS) - jnp.log(std + SEPS))
            + (std * std + (mu - pmu) ** 2) / (pstd * pstd + SEPS) - 1.0)
    row = lax.broadcasted_iota(jnp.int32, (NB, ZD), 0) + i * NB
    ksum = jnp.sum(jnp.where(row < N, term, 0.0))

    @pl.when(i == 0)
    def _():
        kld_ref[0, 0] = 0.0
    kld_ref[0, 0] += 0.5 * ksum / float(N)


def _tc4_body(t, accg_ref, accq_ref, udg_ref, dinv_ref, zd_ref, *rest):
    i = pl.program_id(0)
    d1 = dinv_ref[0][:, :1]
    g = d1 * (accg_ref[0, 0] + accg_ref[1, 0] + udg_ref[...])
    q = accq_ref[0, 0] + accq_ref[1, 0]
    sig = jax.nn.sigmoid(g[:, :HD])
    if t == 0:
        c1_ref, sle_ref = rest
        c1_ref[...] = sig * jnp.tanh(g[:, HD:])
    else:
        c1in_ref, h_ref, sle_ref = rest
        h_ref[...] = sig * jnp.tanh(c1in_ref[...])

    @pl.when(i == 0)
    def _():
        sle_ref[0, 0] = 0.0
    sle_ref[0, 0] += jnp.sum(q * zd_ref[...])


def _dec_body(zi_ref, zj_ref, s1_ref):
    i = pl.program_id(0)
    j = pl.program_id(1)

    @pl.when((i == 0) & (j == 0))
    def _():
        s1_ref[0, 0] = 0.0

    @pl.when(j >= i)
    def _():
        lg = lax.dot_general(zi_ref[...], zj_ref[...],
                             (((1,), (1,)), ((), ())),
                             preferred_element_type=jnp.float32)
        v = jnp.sum(_softplus_sum(lg))
        s1_ref[0, 0] += jnp.where(i == j, v, 2.0 * v)


def _np_spec(f, t):
    return pl.BlockSpec((1, NB, f), lambda i, _t=t: (_t, i, 0))


def _acc_spec(f, t):
    return pl.BlockSpec((2, 1, NB, f), lambda i, _t=t: (0, _t, i, 0))


def _flat_spec(f):
    return pl.BlockSpec((NB, f), lambda i: (i, 0))


def _full1(shape):
    nd = len(shape)
    return pl.BlockSpec(shape, lambda i, _s=nd: (0,) * _s)


_SMEM1 = pl.BlockSpec((1, 1), lambda *_: (0, 0), memory_space=pltpu.SMEM)
_F32 = jnp.float32


def kernel(x, edge_idx_list, params):
    p = params
    rkey = jax.random.key(42)
    eps = jnp.stack([
        jax.random.normal(jax.random.fold_in(rkey, 2 * t), (N, ZD),
                          _F32) for t in range(T)])
    msk = jnp.stack([
        jax.random.bernoulli(jax.random.fold_in(rkey, 2 * t + 1), 0.5,
                             (N, ZD)) for t in range(T)]).astype(_F32)
    eps = jnp.pad(eps, ((0, 0), (0, NP - N), (0, 0)))
    msk = jnp.pad(msk, ((0, 0), (0, NP - N), (0, 0)))
    xp = jnp.pad(x, ((0, 0), (0, NP - N), (0, 0)))

    pr = jax.nn.relu(p['prior_b'])
    pmu = (pr @ p['prior_mu_W'] + p['prior_mu_b']).reshape(1, ZD)
    pstd = jax.nn.softplus(pr @ p['prior_lv_W']
                           + p['prior_lv_b']).reshape(1, ZD)
    encw = p['enc_W'][:HD]
    wc = jnp.concatenate([p['enc_mu_W'], p['enc_lv_W']], axis=1)
    wd0 = jnp.concatenate([p['Wxi'], p['Wxc']], axis=1)
    wd1 = jnp.concatenate([p['Wxo'], p['Wxc']], axis=1)
    wda = jnp.stack([wd0[:HD], wd1[:HD]])
    wdb = jnp.stack([wd0[HD:], wd1[HD:]])
    bx = p['phi_x_b'].reshape(1, HD)
    bz = p['phi_z_b'].reshape(1, HD)

    ei = edge_idx_list
    row = ei[:, 0].reshape(T, NW, NCH, CH)
    col = ei[:, 1].reshape(T, NW, NCH, CH)

    (degp,) = _get_sc_pass((16,), T, False)(row, row)

    phi, ub, dinv = pl.pallas_call(
        _tc1_body,
        grid=(T, TB),
        in_specs=[pl.BlockSpec((1, NB, XD), lambda t, i: (t, i, 0)),
                  pl.BlockSpec((2, 1, NB, 16), lambda t, i: (0, t, i, 0)),
                  pl.BlockSpec((XD, HD), lambda t, i: (0, 0)),
                  pl.BlockSpec((1, HD), lambda t, i: (0, 0)),
                  pl.BlockSpec((HD, HD), lambda t, i: (0, 0))],
        out_specs=[pl.BlockSpec((1, NB, HD), lambda t, i: (t, i, 0)),
                   pl.BlockSpec((1, NB, HD), lambda t, i: (t, i, 0)),
                   pl.BlockSpec((1, NB, 16), lambda t, i: (t, i, 0))],
        out_shape=[jax.ShapeDtypeStruct((T, NP, HD), _F32),
                   jax.ShapeDtypeStruct((T, NP, HD), _F32),
                   jax.ShapeDtypeStruct((T, NP, 16), _F32)],
    )(xp, degp, p['phi_x_W'], bx, encw)

    sc_b = _get_sc_pass((HD,), 1, True)
    sc_c = sc_b
    accb, uc, accc, mu_t, zd_t, udg_t, kld_t, s1_t = ([None] * T
        for _ in range(8))
    for t in range(T):
        (accb[t],) = sc_b(ub[t], row[t:t + 1], col[t:t + 1])
    for t in range(T):
        (uc[t],) = pl.pallas_call(
            functools.partial(_tc2_body, t),
            grid=(TB,),
            in_specs=[_acc_spec(HD, 0), _np_spec(HD, t), _np_spec(16, t),
                      _full1((HD, HD))],
            out_specs=[_flat_spec(HD)],
            out_shape=[jax.ShapeDtypeStruct((NP, HD), _F32)],
        )(accb[t], ub, dinv, wc)
        (accc[t],) = sc_c(uc[t], row[t:t + 1], col[t:t + 1])
    for t in range(T):
        mu_t[t], zd_t[t], udg_t[t], kld_t[t] = pl.pallas_call(
            functools.partial(_tc3_body, t),
            grid=(TB,),
            in_specs=[_acc_spec(HD, 0), _flat_spec(HD), _np_spec(16, t),
                      _np_spec(ZD, t), _np_spec(ZD, t), _np_spec(HD, t),
                      _full1((ZD, HD)), _full1((1, HD)),
                      pl.BlockSpec((1, HD, 2 * HD), lambda i, _t=t: (_t, 0, 0)),
                      pl.BlockSpec((1, HD, 2 * HD), lambda i, _t=t: (_t, 0, 0)),
                      _full1((1, ZD)), _full1((1, ZD))],
            out_specs=[_flat_spec(ZD), _flat_spec(ZD), _flat_spec(2 * HD),
                       _SMEM1],
            out_shape=[jax.ShapeDtypeStruct((NP, ZD), _F32),
                       jax.ShapeDtypeStruct((NP, ZD), _F32),
                       jax.ShapeDtypeStruct((NP, 2 * HD), _F32),
                       jax.ShapeDtypeStruct((1, 1), _F32)],
        )(accc[t], uc[t], dinv, eps, msk, phi, p['phi_z_W'], bz, wda, wdb,
          pmu, pstd)

    accg, accq = _get_sc_pass((2 * HD, ZD), T, True)(
        udg_t[0], udg_t[1], zd_t[0], zd_t[1], row, col)

    for t in range(T):
        (s1_t[t],) = pl.pallas_call(
            _dec_body,
            grid=(DB, DB),
            in_specs=[pl.BlockSpec((BM, ZD), lambda i, j: (i, 0)),
                      pl.BlockSpec((BM, ZD), lambda i, j: (j, 0))],
            out_specs=[pl.BlockSpec((1, 1), lambda i, j: (0, 0),
                                    memory_space=pltpu.SMEM)],
            out_shape=[jax.ShapeDtypeStruct((1, 1), _F32)],
        )(zd_t[t], zd_t[t])

    c1, sle0 = pl.pallas_call(
        functools.partial(_tc4_body, 0),
        grid=(TB,),
        in_specs=[_acc_spec(2 * HD, 0), _acc_spec(ZD, 0), _flat_spec(2 * HD),
                  _np_spec(16, 0), _flat_spec(ZD)],
        out_specs=[_flat_spec(HD), _SMEM1],
        out_shape=[jax.ShapeDtypeStruct((NP, HD), _F32),
                   jax.ShapeDtypeStruct((1, 1), _F32)],
    )(accg, accq, udg_t[0], dinv, zd_t[0])
    h_out, sle1 = pl.pallas_call(
        functools.partial(_tc4_body, 1),
        grid=(TB,),
        in_specs=[_acc_spec(2 * HD, 1), _acc_spec(ZD, 1), _flat_spec(2 * HD),
                  _np_spec(16, 1), _flat_spec(ZD), _flat_spec(HD)],
        out_specs=[_flat_spec(HD), _SMEM1],
        out_shape=[jax.ShapeDtypeStruct((NP, HD), _F32),
                   jax.ShapeDtypeStruct((1, 1), _F32)],
    )(accg, accq, udg_t[1], dinv, zd_t[1], c1)

    padc = float(NP * NP - N * N)
    sp0 = jnp.log(1.0 + jnp.exp(_F32(0.0)))
    nll = ((s1_t[0][0, 0] - padc * sp0 - sle0[0, 0])
           + (s1_t[1][0, 0] - padc * sp0 - sle1[0, 0])) / float(N * N)
    kld_s = kld_t[0][0, 0] + kld_t[1][0, 0]
    mus = jnp.stack([mu_t[0][:N], mu_t[1][:N]])
    h = h_out[None, :N, :]
    return kld_s, nll, mus, h


# speedup vs baseline: 1.0254x; 1.0254x over previous
import functools

import jax
import jax.numpy as jnp
from jax import lax
from jax.experimental import pallas as pl
from jax.experimental.pallas import tpu as pltpu
from jax.experimental.pallas import tpu_sc as plsc

XD = 128
HD = 32
ZD = 16
T = 2
N = 10000
E = 320000
NP = 10240
NB = 2048
TB = NP // NB
BM = 2048
DB = NP // BM
NC = 2
NS = 16
NW = NC * NS
EPW = E // NW
CH = 80
NCH = EPW // CH
KB = 5
STRIPE = NP // NS
SEPS = 1e-8


def _softplus(v):
    return jnp.maximum(v, 0.0) + jnp.log1p(jnp.exp(-jnp.abs(v)))


def _softplus_sum(v):
    return jnp.maximum(v, 0.0) + jnp.log(1.0 + jnp.exp(-jnp.abs(v)))


def _make_sc_pass(fs, nt, with_gather, kb=KB):
    mesh = plsc.VectorSubcoreMesh(core_axis_name="c", subcore_axis_name="s",
                                  num_cores=NC, num_subcores=NS)
    out_type = tuple(
        jax.ShapeDtypeStruct((NC, nt, NP, f), jnp.float32) for f in fs)
    scratch = [
        pltpu.VMEM((NCH, CH), jnp.int32),
        pltpu.VMEM((NCH, CH), jnp.int32),
    ]
    nbuf = kb if with_gather else 1
    for f in fs:
        for _ in range(nbuf):
            scratch.append(pltpu.VMEM((CH, f), jnp.float32))
            scratch.append(pltpu.SemaphoreType.DMA)
        for _ in range(kb):
            scratch.append(pltpu.SemaphoreType.DMA)
        scratch.append(pltpu.VMEM_SHARED((NP, f), jnp.float32))

    @functools.partial(
        pl.kernel, out_type=out_type, mesh=mesh, scratch_types=scratch,
        compiler_params=pltpu.CompilerParams(use_tc_tiling_on_sc=False))
    def k(*refs):
        nu = len(fs)
        ng = nu * nt if with_gather else 0
        us = refs[:ng]
        gidx_hbm = refs[ng]
        sidx_hbm = refs[ng + 1]
        outs = refs[ng + 2:ng + 2 + nu]
        sc = refs[ng + 2 + nu:]
        gi_v, si_v = sc[0], sc[1]
        per_u = 2 * nbuf + kb + 1
        rows = []
        sems = []
        ssems = []
        accs = []
        for i in range(nu):
            grp = sc[2 + i * per_u:2 + (i + 1) * per_u]
            rows.append([grp[2 * k] for k in range(nbuf)])
            sems.append([grp[2 * k + 1] for k in range(nbuf)])
            ssems.append([grp[2 * nbuf + k] for k in range(kb)])
            accs.append(grp[2 * nbuf + kb])

        cid = lax.axis_index("c")
        sid = lax.axis_index("s")
        wid = sid * NC + cid

        def _fill_rows(val):
            for i, f in enumerate(fs):
                def frow(j, _, _r=rows[i][0], _f=f, _v=val):
                    for kk in range(_f // 16):
                        _r[j, pl.ds(16 * kk, 16)] = jnp.full(
                            (16,), _v, jnp.float32)
                    return 0
                lax.fori_loop(0, CH, frow, 0)

        def _start_gather(i, t, k, j):
            pltpu.async_copy(us[i * nt + t].at[gi_v.at[j]], rows[i][k],
                             sems[i][k])

        def _wait_gather(i, t, k):
            pltpu.make_async_copy(us[i * nt + t].at[pl.ds(0, CH)],
                                  rows[i][k], sems[i][k]).wait()

        def _start_scatter(i, b, k, j):
            pltpu.async_copy(rows[i][b], accs[i].at[si_v.at[j]],
                             ssems[i][k], add=True)

        def _wait_scatter(i, b, k):
            pltpu.make_async_copy(rows[i][b], accs[i].at[pl.ds(0, CH)],
                                  ssems[i][k]).wait()

        for t in range(nt):
            _fill_rows(0.0)
            for i in range(nu):
                for kk in range(STRIPE // CH):
                    pltpu.sync_copy(
                        rows[i][0],
                        accs[i].at[pl.ds(sid * STRIPE + kk * CH, CH)])
            if not with_gather:
                _fill_rows(1.0)
            plsc.subcore_barrier()
            pltpu.sync_copy(gidx_hbm.at[t, wid], gi_v)
            pltpu.sync_copy(sidx_hbm.at[t, wid], si_v)

            if with_gather:
                for k in range(kb):
                    for i in range(nu):
                        _start_gather(i, t, k, k)

                def group(g, _, _t=t):
                    for k in range(kb):
                        j = g * kb + k
                        for i in range(nu):
                            _wait_gather(i, _t, k)
                            _start_scatter(i, k, k, j)
                        pk = (k - 1) % kb
                        pj = j - 1 + kb
                        cond = (pj < NCH) if k >= 1 else (
                            (g >= 1) & (pj < NCH))

                        @pl.when(cond)
                        def _(_pk=pk, _pj=pj, _tt=_t):
                            for i in range(nu):
                                _wait_scatter(i, _pk, _pk)
                                _start_gather(i, _tt, _pk, _pj)
                    return 0
                lax.fori_loop(0, NCH // kb, group, 0)
                for k in range(kb):
                    for i in range(nu):
                        _wait_scatter(i, k, k)
            else:
                def group0(g, _):
                    for k in range(kb):
                        j = g * kb + k

                        @pl.when(g >= 1)
                        def _(_k=k):
                            for i in range(nu):
                                _wait_scatter(i, 0, _k)
                        for i in range(nu):
                            _start_scatter(i, 0, k, j)
                    return 0
                lax.fori_loop(0, NCH // kb, group0, 0)
                for k in range(kb):
                    for i in range(nu):
                        _wait_scatter(i, 0, k)
            plsc.subcore_barrier()
            for i in range(nu):
                pltpu.sync_copy(
                    accs[i].at[pl.ds(sid * STRIPE, STRIPE)],
                    outs[i].at[cid, t, pl.ds(sid * STRIPE, STRIPE)])
            plsc.subcore_barrier()

    return k


@functools.lru_cache(maxsize=None)
def _get_sc_pass(fs_key, nt, with_gather, kb=KB):
    return _make_sc_pass(list(fs_key), nt, with_gather, kb)


def _tc0_body(x_ref, w_ref, b_ref, encw_ref, phi_ref, ue_ref):
    phi = jnp.maximum(
        jnp.dot(x_ref[0], w_ref[...],
                preferred_element_type=jnp.float32) + b_ref[...], 0.0)
    phi_ref[0] = phi
    ue_ref[0] = jnp.dot(phi, encw_ref[...],
                        preferred_element_type=jnp.float32)


def _tc1_body(degp_ref, ue_ref, ub_ref, dinv_ref):
    deg = degp_ref[0, 0] + degp_ref[1, 0] + 1.0
    dinv = lax.rsqrt(deg)
    dinv_ref[0] = dinv
    ub_ref[0] = dinv[:, :1] * ue_ref[0]


def _tc2_body(t, accb_ref, ub_ref, dinv_ref, wc_ref, uc_ref):
    d1 = dinv_ref[0][:, :1]
    enc = jnp.maximum(d1 * (accb_ref[0, 0] + accb_ref[1, 0] + ub_ref[0]), 0.0)
    uc_ref[...] = d1 * jnp.dot(enc, wc_ref[...],
                               preferred_element_type=jnp.float32)


def _tc3_body(t, accc_ref, uc_ref, dinv_ref, eps_ref, msk_ref, phi_ref,
              wz_ref, bz_ref, wda_ref, wdb_ref, pmu_ref, pstd_ref,
              mu_ref, zd_ref, udg_ref, kld_ref):
    i = pl.program_id(0)
    d1 = dinv_ref[0][:, :1]
    musd = d1 * (accc_ref[0, 0] + accc_ref[1, 0] + uc_ref[...])
    mu = musd[:, :ZD]
    std = _softplus(musd[:, ZD:])
    mu_ref[...] = mu
    z = mu + eps_ref[0] * std
    phiz = jnp.maximum(
        jnp.dot(z, wz_ref[...], preferred_element_type=jnp.float32)
        + bz_ref[...], 0.0)
    zd = msk_ref[0] * (2.0 * z)
    zd_ref[...] = zd
    udg_ref[...] = d1 * (
        jnp.dot(phi_ref[0], wda_ref[0], preferred_element_type=jnp.float32)
        + jnp.dot(phiz, wdb_ref[0], preferred_element_type=jnp.float32))
    pmu = pmu_ref[...]
    pstd = pstd_ref[...]
    term = (2.0 * (jnp.log(pstd + SEPS) - jnp.log(std + SEPS))
            + (std * std + (mu - pmu) ** 2) / (pstd * pstd + SEPS) - 1.0)
    row = lax.broadcasted_iota(jnp.int32, (NB, ZD), 0) + i * NB
    ksum = jnp.sum(jnp.where(row < N, term, 0.0))

    @pl.when(i == 0)
    def _():
        kld_ref[0, 0] = 0.0
    kld_ref[0, 0] += 0.5 * ksum / float(N)


def _tc4_body(accg_ref, accq_ref, udg0_ref, udg1_ref, dinv_ref,
              zd0_ref, zd1_ref, h_ref, sle_ref, c1_s):
    t = pl.program_id(0)
    i = pl.program_id(1)
    d1 = dinv_ref[0][:, :1]
    q = accq_ref[0, 0] + accq_ref[1, 0]

    @pl.when(i == 0)
    def _():
        sle_ref[0, t] = 0.0

    @pl.when(t == 0)
    def _():
        g = d1 * (accg_ref[0, 0] + accg_ref[1, 0] + udg0_ref[...])
        c1_s[pl.ds(i * NB, NB), :] = (jax.nn.sigmoid(g[:, :HD])
                                      * jnp.tanh(g[:, HD:]))
        sle_ref[0, 0] += jnp.sum(q * zd0_ref[...])

    @pl.when(t == 1)
    def _():
        g = d1 * (accg_ref[0, 0] + accg_ref[1, 0] + udg1_ref[...])
        h_ref[...] = (jax.nn.sigmoid(g[:, :HD])
                      * jnp.tanh(c1_s[pl.ds(i * NB, NB), :]))
        sle_ref[0, 1] += jnp.sum(q * zd1_ref[...])


def _dec_body(zi_ref, zj_ref, s1_ref):
    i = pl.program_id(0)
    j = pl.program_id(1)

    @pl.when((i == 0) & (j == 0))
    def _():
        s1_ref[0, 0] = 0.0

    @pl.when(j >= i)
    def _():
        lg = lax.dot_general(zi_ref[...], zj_ref[...],
                             (((1,), (1,)), ((), ())),
                             preferred_element_type=jnp.float32)
        v = jnp.sum(_softplus_sum(lg))
        s1_ref[0, 0] += jnp.where(i == j, v, 2.0 * v)


def _np_spec(f, t):
    return pl.BlockSpec((1, NB, f), lambda i, _t=t: (_t, i, 0))


def _acc_spec(f, t):
    return pl.BlockSpec((2, 1, NB, f), lambda i, _t=t: (0, _t, i, 0))


def _flat_spec(f):
    return pl.BlockSpec((NB, f), lambda i: (i, 0))


def _full1(shape):
    nd = len(shape)
    return pl.BlockSpec(shape, lambda i, _s=nd: (0,) * _s)


_SMEM1 = pl.BlockSpec((1, 1), lambda *_: (0, 0), memory_space=pltpu.SMEM)
_F32 = jnp.float32


def kernel(x, edge_idx_list, params):
    p = params
    rkey = jax.random.key(42)
    eps = jnp.stack([
        jax.random.normal(jax.random.fold_in(rkey, 2 * t), (N, ZD),
                          _F32) for t in range(T)])
    msk = jnp.stack([
        jax.random.bernoulli(jax.random.fold_in(rkey, 2 * t + 1), 0.5,
                             (N, ZD)) for t in range(T)]).astype(_F32)
    eps = jnp.pad(eps, ((0, 0), (0, NP - N), (0, 0)))
    msk = jnp.pad(msk, ((0, 0), (0, NP - N), (0, 0)))
    xp = jnp.pad(x, ((0, 0), (0, NP - N), (0, 0)))

    pr = jax.nn.relu(p['prior_b'])
    pmu = (pr @ p['prior_mu_W'] + p['prior_mu_b']).reshape(1, ZD)
    pstd = jax.nn.softplus(pr @ p['prior_lv_W']
                           + p['prior_lv_b']).reshape(1, ZD)
    encw = p['enc_W'][:HD]
    wc = jnp.concatenate([p['enc_mu_W'], p['enc_lv_W']], axis=1)
    wd0 = jnp.concatenate([p['Wxi'], p['Wxc']], axis=1)
    wd1 = jnp.concatenate([p['Wxo'], p['Wxc']], axis=1)
    wda = jnp.stack([wd0[:HD], wd1[:HD]])
    wdb = jnp.stack([wd0[HD:], wd1[HD:]])
    bx = p['phi_x_b'].reshape(1, HD)
    bz = p['phi_z_b'].reshape(1, HD)

    ei = edge_idx_list
    row = ei[:, 0].reshape(T, NW, NCH, CH)
    col = ei[:, 1].reshape(T, NW, NCH, CH)

    (degp,) = _get_sc_pass((16,), T, False)(row, row)
    phi, ue = pl.pallas_call(
        _tc0_body,
        grid=(T, TB),
        in_specs=[pl.BlockSpec((1, NB, XD), lambda t, i: (t, i, 0)),
                  pl.BlockSpec((XD, HD), lambda t, i: (0, 0)),
                  pl.BlockSpec((1, HD), lambda t, i: (0, 0)),
                  pl.BlockSpec((HD, HD), lambda t, i: (0, 0))],
        out_specs=[pl.BlockSpec((1, NB, HD), lambda t, i: (t, i, 0)),
                   pl.BlockSpec((1, NB, HD), lambda t, i: (t, i, 0))],
        out_shape=[jax.ShapeDtypeStruct((T, NP, HD), _F32),
                   jax.ShapeDtypeStruct((T, NP, HD), _F32)],
    )(xp, p['phi_x_W'], bx, encw)

    ub, dinv = pl.pallas_call(
        _tc1_body,
        grid=(T, TB),
        in_specs=[pl.BlockSpec((2, 1, NB, 16), lambda t, i: (0, t, i, 0)),
                  pl.BlockSpec((1, NB, HD), lambda t, i: (t, i, 0))],
        out_specs=[pl.BlockSpec((1, NB, HD), lambda t, i: (t, i, 0)),
                   pl.BlockSpec((1, NB, 16), lambda t, i: (t, i, 0))],
        out_shape=[jax.ShapeDtypeStruct((T, NP, HD), _F32),
                   jax.ShapeDtypeStruct((T, NP, 16), _F32)],
    )(degp, ue)

    sc_b = _get_sc_pass((HD,), 1, True)
    sc_c = sc_b
    accb, uc, accc, mu_t, zd_t, udg_t, kld_t, s1_t = ([None] * T
        for _ in range(8))
    for t in range(T):
        (accb[t],) = sc_b(ub[t], row[t:t + 1], col[t:t + 1])
    for t in range(T):
        (uc[t],) = pl.pallas_call(
            functools.partial(_tc2_body, t),
            grid=(TB,),
            in_specs=[_acc_spec(HD, 0), _np_spec(HD, t), _np_spec(16, t),
                      _full1((HD, HD))],
            out_specs=[_flat_spec(HD)],
            out_shape=[jax.ShapeDtypeStruct((NP, HD), _F32)],
        )(accb[t], ub, dinv, wc)
        (accc[t],) = sc_c(uc[t], row[t:t + 1], col[t:t + 1])
    for t in range(T):
        mu_t[t], zd_t[t], udg_t[t], kld_t[t] = pl.pallas_call(
            functools.partial(_tc3_body, t),
            grid=(TB,),
            in_specs=[_acc_spec(HD, 0), _flat_spec(HD), _np_spec(16, t),
                      _np_spec(ZD, t), _np_spec(ZD, t), _np_spec(HD, t),
                      _full1((ZD, HD)), _full1((1, HD)),
                      pl.BlockSpec((1, HD, 2 * HD), lambda i, _t=t: (_t, 0, 0)),
                      pl.BlockSpec((1, HD, 2 * HD), lambda i, _t=t: (_t, 0, 0)),
                      _full1((1, ZD)), _full1((1, ZD))],
            out_specs=[_flat_spec(ZD), _flat_spec(ZD), _flat_spec(2 * HD),
                       _SMEM1],
            out_shape=[jax.ShapeDtypeStruct((NP, ZD), _F32),
                       jax.ShapeDtypeStruct((NP, ZD), _F32),
                       jax.ShapeDtypeStruct((NP, 2 * HD), _F32),
                       jax.ShapeDtypeStruct((1, 1), _F32)],
        )(accc[t], uc[t], dinv, eps, msk, phi, p['phi_z_W'], bz, wda, wdb,
          pmu, pstd)

    accg, accq = _get_sc_pass((2 * HD, ZD), T, True)(
        udg_t[0], udg_t[1], zd_t[0], zd_t[1], row, col)

    for t in range(T):
        (s1_t[t],) = pl.pallas_call(
            _dec_body,
            grid=(DB, DB),
            in_specs=[pl.BlockSpec((BM, ZD), lambda i, j: (i, 0)),
                      pl.BlockSpec((BM, ZD), lambda i, j: (j, 0))],
            out_specs=[pl.BlockSpec((1, 1), lambda i, j: (0, 0),
                                    memory_space=pltpu.SMEM)],
            out_shape=[jax.ShapeDtypeStruct((1, 1), _F32)],
        )(zd_t[t], zd_t[t])

    h_out, sle = pl.pallas_call(
        _tc4_body,
        grid=(T, TB),
        in_specs=[pl.BlockSpec((2, 1, NB, 2 * HD),
                               lambda t, i: (0, t, i, 0)),
                  pl.BlockSpec((2, 1, NB, ZD), lambda t, i: (0, t, i, 0)),
                  pl.BlockSpec((NB, 2 * HD), lambda t, i: (i, 0)),
                  pl.BlockSpec((NB, 2 * HD), lambda t, i: (i, 0)),
                  pl.BlockSpec((1, NB, 16), lambda t, i: (t, i, 0)),
                  pl.BlockSpec((NB, ZD), lambda t, i: (i, 0)),
                  pl.BlockSpec((NB, ZD), lambda t, i: (i, 0))],
        out_specs=[pl.BlockSpec((NB, HD), lambda t, i: (i, 0)),
                   pl.BlockSpec((1, T), lambda t, i: (0, 0),
                                memory_space=pltpu.SMEM)],
        out_shape=[jax.ShapeDtypeStruct((NP, HD), _F32),
                   jax.ShapeDtypeStruct((1, T), _F32)],
        scratch_shapes=[pltpu.VMEM((NP, HD), _F32)],
    )(accg, accq, udg_t[0], udg_t[1], dinv, zd_t[0], zd_t[1])

    padc = float(NP * NP - N * N)
    sp0 = jnp.log(1.0 + jnp.exp(_F32(0.0)))
    nll = ((s1_t[0][0, 0] - padc * sp0 - sle[0, 0])
           + (s1_t[1][0, 0] - padc * sp0 - sle[0, 1])) / float(N * N)
    kld_s = kld_t[0][0, 0] + kld_t[1][0, 0]
    mus = jnp.stack([mu_t[0][:N], mu_t[1][:N]])
    h = h_out[None, :N, :]
    return kld_s, nll, mus, h


# speedup vs baseline: 1.0425x; 1.0167x over previous
import functools

import jax
import jax.numpy as jnp
from jax import lax
from jax.experimental import pallas as pl
from jax.experimental.pallas import tpu as pltpu
from jax.experimental.pallas import tpu_sc as plsc

XD = 128
HD = 32
ZD = 16
T = 2
N = 10000
E = 320000
NP = 10240
NB = 2048
TB = NP // NB
BM = 2048
DB = NP // BM
NC = 2
NS = 16
NW = NC * NS
EPW = E // NW
CH = 80
NCH = EPW // CH
KB = 5
STRIPE = NP // NS
SEPS = 1e-8


def _softplus(v):
    return jnp.maximum(v, 0.0) + jnp.log1p(jnp.exp(-jnp.abs(v)))


def _softplus_sum(v):
    return jnp.maximum(v, 0.0) + jnp.log(1.0 + jnp.exp(-jnp.abs(v)))


def _make_sc_pass(fs, nt, with_gather, kb=KB):
    mesh = plsc.VectorSubcoreMesh(core_axis_name="c", subcore_axis_name="s",
                                  num_cores=NC, num_subcores=NS)
    out_type = tuple(
        jax.ShapeDtypeStruct((NC, nt, NP, f), jnp.float32) for f in fs)
    scratch = [
        pltpu.VMEM((NCH, CH), jnp.int32),
        pltpu.VMEM((NCH, CH), jnp.int32),
    ]
    nbuf = kb if with_gather else 1
    for f in fs:
        for _ in range(nbuf):
            scratch.append(pltpu.VMEM((CH, f), jnp.float32))
            scratch.append(pltpu.SemaphoreType.DMA)
        for _ in range(kb):
            scratch.append(pltpu.SemaphoreType.DMA)
        scratch.append(pltpu.VMEM_SHARED((NP, f), jnp.float32))

    @functools.partial(
        pl.kernel, out_type=out_type, mesh=mesh, scratch_types=scratch,
        compiler_params=pltpu.CompilerParams(use_tc_tiling_on_sc=False))
    def k(*refs):
        nu = len(fs)
        ng = nu * nt if with_gather else 0
        us = refs[:ng]
        gidx_hbm = refs[ng]
        sidx_hbm = refs[ng + 1]
        outs = refs[ng + 2:ng + 2 + nu]
        sc = refs[ng + 2 + nu:]
        gi_v, si_v = sc[0], sc[1]
        per_u = 2 * nbuf + kb + 1
        rows = []
        sems = []
        ssems = []
        accs = []
        for i in range(nu):
            grp = sc[2 + i * per_u:2 + (i + 1) * per_u]
            rows.append([grp[2 * k] for k in range(nbuf)])
            sems.append([grp[2 * k + 1] for k in range(nbuf)])
            ssems.append([grp[2 * nbuf + k] for k in range(kb)])
            accs.append(grp[2 * nbuf + kb])

        cid = lax.axis_index("c")
        sid = lax.axis_index("s")
        wid = sid * NC + cid

        def _fill_rows(val):
            for i, f in enumerate(fs):
                def frow(j, _, _r=rows[i][0], _f=f, _v=val):
                    for kk in range(_f // 16):
                        _r[j, pl.ds(16 * kk, 16)] = jnp.full(
                            (16,), _v, jnp.float32)
                    return 0
                lax.fori_loop(0, CH, frow, 0)

        def _start_gather(i, t, k, j):
            pltpu.async_copy(us[i * nt + t].at[gi_v.at[j]], rows[i][k],
                             sems[i][k])

        def _wait_gather(i, t, k):
            pltpu.make_async_copy(us[i * nt + t].at[pl.ds(0, CH)],
                                  rows[i][k], sems[i][k]).wait()

        def _start_scatter(i, b, k, j):
            pltpu.async_copy(rows[i][b], accs[i].at[si_v.at[j]],
                             ssems[i][k], add=True)

        def _wait_scatter(i, b, k):
            pltpu.make_async_copy(rows[i][b], accs[i].at[pl.ds(0, CH)],
                                  ssems[i][k]).wait()

        for t in range(nt):
            _fill_rows(0.0)
            for i in range(nu):
                for kk in range(STRIPE // CH):
                    pltpu.sync_copy(
                        rows[i][0],
                        accs[i].at[pl.ds(sid * STRIPE + kk * CH, CH)])
            if not with_gather:
                _fill_rows(1.0)
            plsc.subcore_barrier()
            pltpu.sync_copy(gidx_hbm.at[t, wid], gi_v)
            pltpu.sync_copy(sidx_hbm.at[t, wid], si_v)

            if with_gather:
                for k in range(kb):
                    for i in range(nu):
                        _start_gather(i, t, k, k)

                def group(g, _, _t=t):
                    for k in range(kb):
                        j = g * kb + k
                        for i in range(nu):
                            _wait_gather(i, _t, k)
                            _start_scatter(i, k, k, j)
                        pk = (k - 1) % kb
                        pj = j - 1 + kb
                        cond = (pj < NCH) if k >= 1 else (
                            (g >= 1) & (pj < NCH))

                        @pl.when(cond)
                        def _(_pk=pk, _pj=pj, _tt=_t):
                            for i in range(nu):
                                _wait_scatter(i, _pk, _pk)
                                _start_gather(i, _tt, _pk, _pj)
                    return 0
                lax.fori_loop(0, NCH // kb, group, 0)
                for k in range(kb):
                    for i in range(nu):
                        _wait_scatter(i, k, k)
            else:
                def group0(g, _):
                    for k in range(kb):
                        j = g * kb + k

                        @pl.when(g >= 1)
                        def _(_k=k):
                            for i in range(nu):
                                _wait_scatter(i, 0, _k)
                        for i in range(nu):
                            _start_scatter(i, 0, k, j)
                    return 0
                lax.fori_loop(0, NCH // kb, group0, 0)
                for k in range(kb):
                    for i in range(nu):
                        _wait_scatter(i, 0, k)
            plsc.subcore_barrier()
            for i in range(nu):
                pltpu.sync_copy(
                    accs[i].at[pl.ds(sid * STRIPE, STRIPE)],
                    outs[i].at[cid, t, pl.ds(sid * STRIPE, STRIPE)])
            plsc.subcore_barrier()

    return k


@functools.lru_cache(maxsize=None)
def _get_sc_pass(fs_key, nt, with_gather, kb=KB):
    return _make_sc_pass(list(fs_key), nt, with_gather, kb)


def _tc1_body(x_ref, degp_ref, w_ref, b_ref, encw_ref,
              phi_ref, ub_ref, dinv_ref):
    deg = degp_ref[0, 0] + degp_ref[1, 0] + 1.0
    dinv = lax.rsqrt(deg)
    dinv_ref[0] = dinv
    phi = jnp.maximum(
        jnp.dot(x_ref[0], w_ref[...],
                preferred_element_type=jnp.float32) + b_ref[...], 0.0)
    phi_ref[0] = phi
    ub_ref[0] = dinv[:, :1] * jnp.dot(phi, encw_ref[...],
                                      preferred_element_type=jnp.float32)


def _tc2_body(t, accb_ref, ub_ref, dinv_ref, wc_ref, uc_ref):
    d1 = dinv_ref[0][:, :1]
    enc = jnp.maximum(d1 * (accb_ref[0, 0] + accb_ref[1, 0] + ub_ref[0]), 0.0)
    uc_ref[...] = d1 * jnp.dot(enc, wc_ref[...],
                               preferred_element_type=jnp.float32)


def _tc3_body(t, accc_ref, uc_ref, dinv_ref, eps_ref, msk_ref, phi_ref,
              wz_ref, bz_ref, wda_ref, wdb_ref, pmu_ref, pstd_ref,
              mu_ref, zd_ref, udg_ref, kld_ref):
    i = pl.program_id(0)
    d1 = dinv_ref[0][:, :1]
    musd = d1 * (accc_ref[0, 0] + accc_ref[1, 0] + uc_ref[...])
    mu = musd[:, :ZD]
    std = _softplus(musd[:, ZD:])
    mu_ref[...] = mu
    z = mu + eps_ref[0] * std
    phiz = jnp.maximum(
        jnp.dot(z, wz_ref[...], preferred_element_type=jnp.float32)
        + bz_ref[...], 0.0)
    zd = msk_ref[0] * (2.0 * z)
    zd_ref[...] = zd
    udg_ref[...] = d1 * (
        jnp.dot(phi_ref[0], wda_ref[0], preferred_element_type=jnp.float32)
        + jnp.dot(phiz, wdb_ref[0], preferred_element_type=jnp.float32))
    pmu = pmu_ref[...]
    pstd = pstd_ref[...]
    term = (2.0 * (jnp.log(pstd + SEPS) - jnp.log(std + SEPS))
            + (std * std + (mu - pmu) ** 2) / (pstd * pstd + SEPS) - 1.0)
    row = lax.broadcasted_iota(jnp.int32, (NB, ZD), 0) + i * NB
    ksum = jnp.sum(jnp.where(row < N, term, 0.0))

    @pl.when(i == 0)
    def _():
        kld_ref[0, 0] = 0.0
    kld_ref[0, 0] += 0.5 * ksum / float(N)


def _tc4_body(accg_ref, accq_ref, udg0_ref, udg1_ref, dinv_ref,
              zd0_ref, zd1_ref, h_ref, sle_ref, c1_s):
    t = pl.program_id(0)
    i = pl.program_id(1)
    d1 = dinv_ref[0][:, :1]
    q = accq_ref[0, 0] + accq_ref[1, 0]

    @pl.when(i == 0)
    def _():
        sle_ref[0, t] = 0.0

    @pl.when(t == 0)
    def _():
        g = d1 * (accg_ref[0, 0] + accg_ref[1, 0] + udg0_ref[...])
        c1_s[pl.ds(i * NB, NB), :] = (jax.nn.sigmoid(g[:, :HD])
                                      * jnp.tanh(g[:, HD:]))
        sle_ref[0, 0] += jnp.sum(q * zd0_ref[...])

    @pl.when(t == 1)
    def _():
        g = d1 * (accg_ref[0, 0] + accg_ref[1, 0] + udg1_ref[...])
        h_ref[...] = (jax.nn.sigmoid(g[:, :HD])
                      * jnp.tanh(c1_s[pl.ds(i * NB, NB), :]))
        sle_ref[0, 1] += jnp.sum(q * zd1_ref[...])


def _dec_body(zi_ref, zj_ref, s1_ref):
    i = pl.program_id(0)
    j = pl.program_id(1)

    @pl.when((i == 0) & (j == 0))
    def _():
        s1_ref[0, 0] = 0.0

    @pl.when(j >= i)
    def _():
        lg = lax.dot_general(zi_ref[...], zj_ref[...],
                             (((1,), (1,)), ((), ())),
                             preferred_element_type=jnp.float32)
        v = jnp.sum(_softplus_sum(lg))
        s1_ref[0, 0] += jnp.where(i == j, v, 2.0 * v)


def _np_spec(f, t):
    return pl.BlockSpec((1, NB, f), lambda i, _t=t: (_t, i, 0))


def _acc_spec(f, t):
    return pl.BlockSpec((2, 1, NB, f), lambda i, _t=t: (0, _t, i, 0))


def _flat_spec(f):
    return pl.BlockSpec((NB, f), lambda i: (i, 0))


def _full1(shape):
    nd = len(shape)
    return pl.BlockSpec(shape, lambda i, _s=nd: (0,) * _s)


_SMEM1 = pl.BlockSpec((1, 1), lambda *_: (0, 0), memory_space=pltpu.SMEM)
_F32 = jnp.float32


def kernel(x, edge_idx_list, params):
    p = params
    rkey = jax.random.key(42)
    eps = jnp.stack([
        jax.random.normal(jax.random.fold_in(rkey, 2 * t), (N, ZD),
                          _F32) for t in range(T)])
    msk = jnp.stack([
        jax.random.bernoulli(jax.random.fold_in(rkey, 2 * t + 1), 0.5,
                             (N, ZD)) for t in range(T)]).astype(_F32)
    eps = jnp.pad(eps, ((0, 0), (0, NP - N), (0, 0)))
    msk = jnp.pad(msk, ((0, 0), (0, NP - N), (0, 0)))
    xp = jnp.pad(x, ((0, 0), (0, NP - N), (0, 0)))

    pr = jax.nn.relu(p['prior_b'])
    pmu = (pr @ p['prior_mu_W'] + p['prior_mu_b']).reshape(1, ZD)
    pstd = jax.nn.softplus(pr @ p['prior_lv_W']
                           + p['prior_lv_b']).reshape(1, ZD)
    encw = p['enc_W'][:HD]
    wc = jnp.concatenate([p['enc_mu_W'], p['enc_lv_W']], axis=1)
    wd0 = jnp.concatenate([p['Wxi'], p['Wxc']], axis=1)
    wd1 = jnp.concatenate([p['Wxo'], p['Wxc']], axis=1)
    wda = jnp.stack([wd0[:HD], wd1[:HD]])
    wdb = jnp.stack([wd0[HD:], wd1[HD:]])
    bx = p['phi_x_b'].reshape(1, HD)
    bz = p['phi_z_b'].reshape(1, HD)

    ei = edge_idx_list
    row = ei[:, 0].reshape(T, NW, NCH, CH)
    col = ei[:, 1].reshape(T, NW, NCH, CH)

    (degp,) = _get_sc_pass((16,), T, False)(row, row)

    phi, ub, dinv = pl.pallas_call(
        _tc1_body,
        grid=(T, TB),
        in_specs=[pl.BlockSpec((1, NB, XD), lambda t, i: (t, i, 0)),
                  pl.BlockSpec((2, 1, NB, 16), lambda t, i: (0, t, i, 0)),
                  pl.BlockSpec((XD, HD), lambda t, i: (0, 0)),
                  pl.BlockSpec((1, HD), lambda t, i: (0, 0)),
                  pl.BlockSpec((HD, HD), lambda t, i: (0, 0))],
        out_specs=[pl.BlockSpec((1, NB, HD), lambda t, i: (t, i, 0)),
                   pl.BlockSpec((1, NB, HD), lambda t, i: (t, i, 0)),
                   pl.BlockSpec((1, NB, 16), lambda t, i: (t, i, 0))],
        out_shape=[jax.ShapeDtypeStruct((T, NP, HD), _F32),
                   jax.ShapeDtypeStruct((T, NP, HD), _F32),
                   jax.ShapeDtypeStruct((T, NP, 16), _F32)],
    )(xp, degp, p['phi_x_W'], bx, encw)

    sc_b = _get_sc_pass((HD,), 1, True)
    sc_c = sc_b
    accb, uc, accc, mu_t, zd_t, udg_t, kld_t, s1_t = ([None] * T
        for _ in range(8))
    for t in range(T):
        (accb[t],) = sc_b(ub[t], row[t:t + 1], col[t:t + 1])
    for t in range(T):
        (uc[t],) = pl.pallas_call(
            functools.partial(_tc2_body, t),
            grid=(TB,),
            in_specs=[_acc_spec(HD, 0), _np_spec(HD, t), _np_spec(16, t),
                      _full1((HD, HD))],
            out_specs=[_flat_spec(HD)],
            out_shape=[jax.ShapeDtypeStruct((NP, HD), _F32)],
        )(accb[t], ub, dinv, wc)
        (accc[t],) = sc_c(uc[t], row[t:t + 1], col[t:t + 1])
    for t in range(T):
        mu_t[t], zd_t[t], udg_t[t], kld_t[t] = pl.pallas_call(
            functools.partial(_tc3_body, t),
            grid=(TB,),
            in_specs=[_acc_spec(HD, 0), _flat_spec(HD), _np_spec(16, t),
                      _np_spec(ZD, t), _np_spec(ZD, t), _np_spec(HD, t),
                      _full1((ZD, HD)), _full1((1, HD)),
                      pl.BlockSpec((1, HD, 2 * HD), lambda i, _t=t: (_t, 0, 0)),
                      pl.BlockSpec((1, HD, 2 * HD), lambda i, _t=t: (_t, 0, 0)),
                      _full1((1, ZD)), _full1((1, ZD))],
            out_specs=[_flat_spec(ZD), _flat_spec(ZD), _flat_spec(2 * HD),
                       _SMEM1],
            out_shape=[jax.ShapeDtypeStruct((NP, ZD), _F32),
                       jax.ShapeDtypeStruct((NP, ZD), _F32),
                       jax.ShapeDtypeStruct((NP, 2 * HD), _F32),
                       jax.ShapeDtypeStruct((1, 1), _F32)],
        )(accc[t], uc[t], dinv, eps, msk, phi, p['phi_z_W'], bz, wda, wdb,
          pmu, pstd)

    accg, accq = _get_sc_pass((2 * HD, ZD), T, True)(
        udg_t[0], udg_t[1], zd_t[0], zd_t[1], row, col)

    for t in range(T):
        (s1_t[t],) = pl.pallas_call(
            _dec_body,
            grid=(DB, DB),
            in_specs=[pl.BlockSpec((BM, ZD), lambda i, j: (i, 0)),
                      pl.BlockSpec((BM, ZD), lambda i, j: (j, 0))],
            out_specs=[pl.BlockSpec((1, 1), lambda i, j: (0, 0),
                                    memory_space=pltpu.SMEM)],
            out_shape=[jax.ShapeDtypeStruct((1, 1), _F32)],
        )(zd_t[t], zd_t[t])

    h_out, sle = pl.pallas_call(
        _tc4_body,
        grid=(T, TB),
        in_specs=[pl.BlockSpec((2, 1, NB, 2 * HD),
                               lambda t, i: (0, t, i, 0)),
                  pl.BlockSpec((2, 1, NB, ZD), lambda t, i: (0, t, i, 0)),
                  pl.BlockSpec((NB, 2 * HD), lambda t, i: (i, 0)),
                  pl.BlockSpec((NB, 2 * HD), lambda t, i: (i, 0)),
                  pl.BlockSpec((1, NB, 16), lambda t, i: (t, i, 0)),
                  pl.BlockSpec((NB, ZD), lambda t, i: (i, 0)),
                  pl.BlockSpec((NB, ZD), lambda t, i: (i, 0))],
        out_specs=[pl.BlockSpec((NB, HD), lambda t, i: (i, 0)),
                   pl.BlockSpec((1, T), lambda t, i: (0, 0),
                                memory_space=pltpu.SMEM)],
        out_shape=[jax.ShapeDtypeStruct((NP, HD), _F32),
                   jax.ShapeDtypeStruct((1, T), _F32)],
        scratch_shapes=[pltpu.VMEM((NP, HD), _F32)],
    )(accg, accq, udg_t[0], udg_t[1], dinv, zd_t[0], zd_t[1])

    padc = float(NP * NP - N * N)
    sp0 = jnp.log(1.0 + jnp.exp(_F32(0.0)))
    nll = ((s1_t[0][0, 0] - padc * sp0 - sle[0, 0])
           + (s1_t[1][0, 0] - padc * sp0 - sle[0, 1])) / float(N * N)
    kld_s = kld_t[0][0, 0] + kld_t[1][0, 0]
    mus = jnp.stack([mu_t[0][:N], mu_t[1][:N]])
    h = h_out[None, :N, :]
    return kld_s, nll, mus, h


# speedup vs baseline: 1.0464x; 1.0038x over previous
import functools

import jax
import jax.numpy as jnp
from jax import lax
from jax.experimental import pallas as pl
from jax.experimental.pallas import tpu as pltpu
from jax.experimental.pallas import tpu_sc as plsc

XD = 128
HD = 32
ZD = 16
T = 2
N = 10000
E = 320000
NP = 10240
NB = 2048
TB = NP // NB
BM = 2048
DB = NP // BM
NC = 2
NS = 16
NW = NC * NS
EPW = E // NW
CH = 80
NCH = EPW // CH
KB = 5
STRIPE = NP // NS
SEPS = 1e-8


def _softplus(v):
    return jnp.maximum(v, 0.0) + jnp.log1p(jnp.exp(-jnp.abs(v)))


def _softplus_sum(v):
    return jnp.maximum(v, 0.0) + jnp.log(1.0 + jnp.exp(-jnp.abs(v)))


def _make_sc_pass(fs, nt, with_gather, kb=KB):
    mesh = plsc.VectorSubcoreMesh(core_axis_name="c", subcore_axis_name="s",
                                  num_cores=NC, num_subcores=NS)
    out_type = tuple(
        jax.ShapeDtypeStruct((NC, nt, NP, f), jnp.float32) for f in fs)
    scratch = [
        pltpu.VMEM((NCH, CH), jnp.int32),
        pltpu.VMEM((NCH, CH), jnp.int32),
    ]
    nbuf = kb if with_gather else 1
    for f in fs:
        for _ in range(nbuf):
            scratch.append(pltpu.VMEM((CH, f), jnp.float32))
            scratch.append(pltpu.SemaphoreType.DMA)
        for _ in range(kb):
            scratch.append(pltpu.SemaphoreType.DMA)
        scratch.append(pltpu.VMEM_SHARED((NP, f), jnp.float32))

    @functools.partial(
        pl.kernel, out_type=out_type, mesh=mesh, scratch_types=scratch,
        compiler_params=pltpu.CompilerParams(use_tc_tiling_on_sc=False))
    def k(*refs):
        nu = len(fs)
        ng = nu * nt if with_gather else 0
        us = refs[:ng]
        gidx_hbm = refs[ng]
        sidx_hbm = refs[ng + 1]
        outs = refs[ng + 2:ng + 2 + nu]
        sc = refs[ng + 2 + nu:]
        gi_v, si_v = sc[0], sc[1]
        per_u = 2 * nbuf + kb + 1
        rows = []
        sems = []
        ssems = []
        accs = []
        for i in range(nu):
            grp = sc[2 + i * per_u:2 + (i + 1) * per_u]
            rows.append([grp[2 * k] for k in range(nbuf)])
            sems.append([grp[2 * k + 1] for k in range(nbuf)])
            ssems.append([grp[2 * nbuf + k] for k in range(kb)])
            accs.append(grp[2 * nbuf + kb])

        cid = lax.axis_index("c")
        sid = lax.axis_index("s")
        wid = sid * NC + cid

        def _fill_rows(val):
            for i, f in enumerate(fs):
                def frow(j, _, _r=rows[i][0], _f=f, _v=val):
                    for kk in range(_f // 16):
                        _r[j, pl.ds(16 * kk, 16)] = jnp.full(
                            (16,), _v, jnp.float32)
                    return 0
                lax.fori_loop(0, CH, frow, 0)

        def _start_gather(i, t, k, j):
            pltpu.async_copy(us[i * nt + t].at[gi_v.at[j]], rows[i][k],
                             sems[i][k])

        def _wait_gather(i, t, k):
            pltpu.make_async_copy(us[i * nt + t].at[pl.ds(0, CH)],
                                  rows[i][k], sems[i][k]).wait()

        def _start_scatter(i, b, k, j):
            pltpu.async_copy(rows[i][b], accs[i].at[si_v.at[j]],
                             ssems[i][k], add=True)

        def _wait_scatter(i, b, k):
            pltpu.make_async_copy(rows[i][b], accs[i].at[pl.ds(0, CH)],
                                  ssems[i][k]).wait()

        for t in range(nt):
            _fill_rows(0.0)
            for i in range(nu):
                for kk in range(STRIPE // CH):
                    pltpu.sync_copy(
                        rows[i][0],
                        accs[i].at[pl.ds(sid * STRIPE + kk * CH, CH)])
            if not with_gather:
                _fill_rows(1.0)
            plsc.subcore_barrier()
            pltpu.sync_copy(gidx_hbm.at[t, wid], gi_v)
            pltpu.sync_copy(sidx_hbm.at[t, wid], si_v)

            if with_gather:
                for k in range(kb):
                    for i in range(nu):
                        _start_gather(i, t, k, k)

                def group(g, _, _t=t):
                    for k in range(kb):
                        j = g * kb + k
                        for i in range(nu):
                            _wait_gather(i, _t, k)
                            _start_scatter(i, k, k, j)
                        pk = (k - 1) % kb
                        pj = j - 1 + kb
                        cond = (pj < NCH) if k >= 1 else (
                            (g >= 1) & (pj < NCH))

                        @pl.when(cond)
                        def _(_pk=pk, _pj=pj, _tt=_t):
                            for i in range(nu):
                                _wait_scatter(i, _pk, _pk)
                                _start_gather(i, _tt, _pk, _pj)
                    return 0
                lax.fori_loop(0, NCH // kb, group, 0)
                for k in range(kb):
                    for i in range(nu):
                        _wait_scatter(i, k, k)
            else:
                def group0(g, _):
                    for k in range(kb):
                        j = g * kb + k

                        @pl.when(g >= 1)
                        def _(_k=k):
                            for i in range(nu):
                                _wait_scatter(i, 0, _k)
                        for i in range(nu):
                            _start_scatter(i, 0, k, j)
                    return 0
                lax.fori_loop(0, NCH // kb, group0, 0)
                for k in range(kb):
                    for i in range(nu):
                        _wait_scatter(i, 0, k)
            plsc.subcore_barrier()
            for i in range(nu):
                pltpu.sync_copy(
                    accs[i].at[pl.ds(sid * STRIPE, STRIPE)],
                    outs[i].at[cid, t, pl.ds(sid * STRIPE, STRIPE)])
            plsc.subcore_barrier()

    return k


@functools.lru_cache(maxsize=None)
def _get_sc_pass(fs_key, nt, with_gather, kb=KB):
    return _make_sc_pass(list(fs_key), nt, with_gather, kb)


def _tc1_body(x_ref, degp_ref, w_ref, b_ref, encw_ref,
              phi_ref, ub_ref, dinv_ref):
    deg = degp_ref[0, 0] + degp_ref[1, 0] + 1.0
    dinv = lax.rsqrt(deg)
    dinv_ref[0] = dinv
    phi = jnp.maximum(
        jnp.dot(x_ref[0], w_ref[...],
                preferred_element_type=jnp.float32) + b_ref[...], 0.0)
    phi_ref[0] = phi
    ub_ref[0] = dinv[:, :1] * jnp.dot(phi, encw_ref[...],
                                      preferred_element_type=jnp.float32)


def _tc2_body(t, accb_ref, ub_ref, dinv_ref, wc_ref, uc_ref):
    d1 = dinv_ref[0][:, :1]
    enc = jnp.maximum(d1 * (accb_ref[0, 0] + accb_ref[1, 0] + ub_ref[0]), 0.0)
    uc_ref[...] = d1 * jnp.dot(enc, wc_ref[...],
                               preferred_element_type=jnp.float32)


def _tc3_body(t, accc_ref, uc_ref, dinv_ref, eps_ref, msk_ref, phi_ref,
              wz_ref, bz_ref, wda_ref, wdb_ref, pmu_ref, pstd_ref,
              mu_ref, zd_ref, udg_ref, kld_ref):
    i = pl.program_id(0)
    d1 = dinv_ref[0][:, :1]
    musd = d1 * (accc_ref[0, 0] + accc_ref[1, 0] + uc_ref[...])
    mu = musd[:, :ZD]
    std = _softplus(musd[:, ZD:])
    mu_ref[...] = mu
    z = mu + eps_ref[0] * std
    phiz = jnp.maximum(
        jnp.dot(z, wz_ref[...], preferred_element_type=jnp.float32)
        + bz_ref[...], 0.0)
    zd = msk_ref[0] * (2.0 * z)
    zd_ref[...] = zd
    udg_ref[...] = d1 * (
        jnp.dot(phi_ref[0], wda_ref[0], preferred_element_type=jnp.float32)
        + jnp.dot(phiz, wdb_ref[0], preferred_element_type=jnp.float32))
    pmu = pmu_ref[...]
    pstd = pstd_ref[...]
    term = (2.0 * (jnp.log(pstd + SEPS) - jnp.log(std + SEPS))
            + (std * std + (mu - pmu) ** 2) / (pstd * pstd + SEPS) - 1.0)
    row = lax.broadcasted_iota(jnp.int32, (NB, ZD), 0) + i * NB
    ksum = jnp.sum(jnp.where(row < N, term, 0.0))

    @pl.when(i == 0)
    def _():
        kld_ref[0, 0] = 0.0
    kld_ref[0, 0] += 0.5 * ksum / float(N)


def _tc4_body(t, accg_ref, accq_ref, udg_ref, dinv_ref, zd_ref, *rest):
    i = pl.program_id(0)
    d1 = dinv_ref[0][:, :1]
    g = d1 * (accg_ref[0, 0] + accg_ref[1, 0] + udg_ref[...])
    q = accq_ref[0, 0] + accq_ref[1, 0]
    sig = jax.nn.sigmoid(g[:, :HD])
    if t == 0:
        c1_ref, sle_ref = rest
        c1_ref[...] = sig * jnp.tanh(g[:, HD:])
    else:
        c1in_ref, h_ref, sle_ref = rest
        h_ref[...] = sig * jnp.tanh(c1in_ref[...])

    @pl.when(i == 0)
    def _():
        sle_ref[0, 0] = 0.0
    sle_ref[0, 0] += jnp.sum(q * zd_ref[...])


def _dec_body(zi_ref, zj_ref, s1_ref):
    i = pl.program_id(0)
    j = pl.program_id(1)

    @pl.when((i == 0) & (j == 0))
    def _():
        s1_ref[0, 0] = 0.0

    @pl.when(j >= i)
    def _():
        lg = lax.dot_general(zi_ref[...], zj_ref[...],
                             (((1,), (1,)), ((), ())),
                             preferred_element_type=jnp.float32)
        v = jnp.sum(_softplus_sum(lg))
        s1_ref[0, 0] += jnp.where(i == j, v, 2.0 * v)


def _np_spec(f, t):
    return pl.BlockSpec((1, NB, f), lambda i, _t=t: (_t, i, 0))


def _acc_spec(f, t):
    return pl.BlockSpec((2, 1, NB, f), lambda i, _t=t: (0, _t, i, 0))


def _flat_spec(f):
    return pl.BlockSpec((NB, f), lambda i: (i, 0))


def _full1(shape):
    nd = len(shape)
    return pl.BlockSpec(shape, lambda i, _s=nd: (0,) * _s)


_SMEM1 = pl.BlockSpec((1, 1), lambda *_: (0, 0), memory_space=pltpu.SMEM)
_F32 = jnp.float32


def kernel(x, edge_idx_list, params):
    p = params
    rkey = jax.random.key(42)
    eps = jnp.stack([
        jax.random.normal(jax.random.fold_in(rkey, 2 * t), (N, ZD),
                          _F32) for t in range(T)])
    msk = jnp.stack([
        jax.random.bernoulli(jax.random.fold_in(rkey, 2 * t + 1), 0.5,
                             (N, ZD)) for t in range(T)]).astype(_F32)
    eps = jnp.pad(eps, ((0, 0), (0, NP - N), (0, 0)))
    msk = jnp.pad(msk, ((0, 0), (0, NP - N), (0, 0)))
    xp = jnp.pad(x, ((0, 0), (0, NP - N), (0, 0)))

    pr = jax.nn.relu(p['prior_b'])
    pmu = (pr @ p['prior_mu_W'] + p['prior_mu_b']).reshape(1, ZD)
    pstd = jax.nn.softplus(pr @ p['prior_lv_W']
                           + p['prior_lv_b']).reshape(1, ZD)
    encw = p['enc_W'][:HD]
    wc = jnp.concatenate([p['enc_mu_W'], p['enc_lv_W']], axis=1)
    wd0 = jnp.concatenate([p['Wxi'], p['Wxc']], axis=1)
    wd1 = jnp.concatenate([p['Wxo'], p['Wxc']], axis=1)
    wda = jnp.stack([wd0[:HD], wd1[:HD]])
    wdb = jnp.stack([wd0[HD:], wd1[HD:]])
    bx = p['phi_x_b'].reshape(1, HD)
    bz = p['phi_z_b'].reshape(1, HD)

    ei = edge_idx_list
    row = ei[:, 0].reshape(T, NW, NCH, CH)
    col = ei[:, 1].reshape(T, NW, NCH, CH)

    (degp,) = _get_sc_pass((16,), T, False)(row, row)

    phi, ub, dinv = pl.pallas_call(
        _tc1_body,
        grid=(T, TB),
        in_specs=[pl.BlockSpec((1, NB, XD), lambda t, i: (t, i, 0)),
                  pl.BlockSpec((2, 1, NB, 16), lambda t, i: (0, t, i, 0)),
                  pl.BlockSpec((XD, HD), lambda t, i: (0, 0)),
                  pl.BlockSpec((1, HD), lambda t, i: (0, 0)),
                  pl.BlockSpec((HD, HD), lambda t, i: (0, 0))],
        out_specs=[pl.BlockSpec((1, NB, HD), lambda t, i: (t, i, 0)),
                   pl.BlockSpec((1, NB, HD), lambda t, i: (t, i, 0)),
                   pl.BlockSpec((1, NB, 16), lambda t, i: (t, i, 0))],
        out_shape=[jax.ShapeDtypeStruct((T, NP, HD), _F32),
                   jax.ShapeDtypeStruct((T, NP, HD), _F32),
                   jax.ShapeDtypeStruct((T, NP, 16), _F32)],
    )(xp, degp, p['phi_x_W'], bx, encw)

    sc_b = _get_sc_pass((HD,), 1, True)
    sc_c = sc_b
    accb, uc, accc, mu_t, zd_t, udg_t, kld_t, s1_t = ([None] * T
        for _ in range(8))
    for t in range(T):
        (accb[t],) = sc_b(ub[t], row[t:t + 1], col[t:t + 1])
    for t in range(T):
        (uc[t],) = pl.pallas_call(
            functools.partial(_tc2_body, t),
            grid=(TB,),
            in_specs=[_acc_spec(HD, 0), _np_spec(HD, t), _np_spec(16, t),
                      _full1((HD, HD))],
            out_specs=[_flat_spec(HD)],
            out_shape=[jax.ShapeDtypeStruct((NP, HD), _F32)],
        )(accb[t], ub, dinv, wc)
        (accc[t],) = sc_c(uc[t], row[t:t + 1], col[t:t + 1])
    for t in range(T):
        mu_t[t], zd_t[t], udg_t[t], kld_t[t] = pl.pallas_call(
            functools.partial(_tc3_body, t),
            grid=(TB,),
            in_specs=[_acc_spec(HD, 0), _flat_spec(HD), _np_spec(16, t),
                      _np_spec(ZD, t), _np_spec(ZD, t), _np_spec(HD, t),
                      _full1((ZD, HD)), _full1((1, HD)),
                      pl.BlockSpec((1, HD, 2 * HD), lambda i, _t=t: (_t, 0, 0)),
                      pl.BlockSpec((1, HD, 2 * HD), lambda i, _t=t: (_t, 0, 0)),
                      _full1((1, ZD)), _full1((1, ZD))],
            out_specs=[_flat_spec(ZD), _flat_spec(ZD), _flat_spec(2 * HD),
                       _SMEM1],
            out_shape=[jax.ShapeDtypeStruct((NP, ZD), _F32),
                       jax.ShapeDtypeStruct((NP, ZD), _F32),
                       jax.ShapeDtypeStruct((NP, 2 * HD), _F32),
                       jax.ShapeDtypeStruct((1, 1), _F32)],
        )(accc[t], uc[t], dinv, eps, msk, phi, p['phi_z_W'], bz, wda, wdb,
          pmu, pstd)

    accg, accq = _get_sc_pass((2 * HD, ZD), T, True)(
        udg_t[0], udg_t[1], zd_t[0], zd_t[1], row, col)

    for t in range(T):
        (s1_t[t],) = pl.pallas_call(
            _dec_body,
            grid=(DB, DB),
            in_specs=[pl.BlockSpec((BM, ZD), lambda i, j: (i, 0)),
                      pl.BlockSpec((BM, ZD), lambda i, j: (j, 0))],
            out_specs=[pl.BlockSpec((1, 1), lambda i, j: (0, 0),
                                    memory_space=pltpu.SMEM)],
            out_shape=[jax.ShapeDtypeStruct((1, 1), _F32)],
        )(zd_t[t], zd_t[t])

    c1, sle0 = pl.pallas_call(
        functools.partial(_tc4_body, 0),
        grid=(TB,),
        in_specs=[_acc_spec(2 * HD, 0), _acc_spec(ZD, 0), _flat_spec(2 * HD),
                  _np_spec(16, 0), _flat_spec(ZD)],
        out_specs=[_flat_spec(HD), _SMEM1],
        out_shape=[jax.ShapeDtypeStruct((NP, HD), _F32),
                   jax.ShapeDtypeStruct((1, 1), _F32)],
    )(accg, accq, udg_t[0], dinv, zd_t[0])
    h_out, sle1 = pl.pallas_call(
        functools.partial(_tc4_body, 1),
        grid=(TB,),
        in_specs=[_acc_spec(2 * HD, 1), _acc_spec(ZD, 1), _flat_spec(2 * HD),
                  _np_spec(16, 1), _flat_spec(ZD), _flat_spec(HD)],
        out_specs=[_flat_spec(HD), _SMEM1],
        out_shape=[jax.ShapeDtypeStruct((NP, HD), _F32),
                   jax.ShapeDtypeStruct((1, 1), _F32)],
    )(accg, accq, udg_t[1], dinv, zd_t[1], c1)

    padc = float(NP * NP - N * N)
    sp0 = jnp.log(1.0 + jnp.exp(_F32(0.0)))
    nll = ((s1_t[0][0, 0] - padc * sp0 - sle0[0, 0])
           + (s1_t[1][0, 0] - padc * sp0 - sle1[0, 0])) / float(N * N)
    kld_s = kld_t[0][0, 0] + kld_t[1][0, 0]
    mus = jnp.stack([mu_t[0][:N], mu_t[1][:N]])
    h = h_out[None, :N, :]
    return kld_s, nll, mus, h


# speedup vs baseline: 1.0719x; 1.0244x over previous
import functools

import jax
import jax.numpy as jnp
from jax import lax
from jax.experimental import pallas as pl
from jax.experimental.pallas import tpu as pltpu
from jax.experimental.pallas import tpu_sc as plsc

XD = 128
HD = 32
ZD = 16
T = 2
N = 10000
E = 320000
NP = 10240
NB = 2048
TB = NP // NB
BM = 2048
DB = NP // BM
NC = 2
NS = 16
NW = NC * NS
EPW = E // NW
CH = 80
NCH = EPW // CH
KB = 5
STRIPE = NP // NS
SEPS = 1e-8


def _softplus(v):
    return jnp.maximum(v, 0.0) + jnp.log1p(jnp.exp(-jnp.abs(v)))


def _softplus_sum(v):
    return jnp.maximum(v, 0.0) + jnp.log(1.0 + jnp.exp(-jnp.abs(v)))


def _make_sc_pass(fs, nt, with_gather, kb=KB, stage_u=False):
    mesh = plsc.VectorSubcoreMesh(core_axis_name="c", subcore_axis_name="s",
                                  num_cores=NC, num_subcores=NS)
    out_type = tuple(
        jax.ShapeDtypeStruct((NC, nt, NP, f), jnp.float32) for f in fs)
    scratch = [
        pltpu.VMEM((NCH, CH), jnp.int32),
        pltpu.VMEM((NCH, CH), jnp.int32),
    ]
    nbuf = kb if with_gather else 1
    for f in fs:
        for _ in range(nbuf):
            scratch.append(pltpu.VMEM((CH, f), jnp.float32))
            scratch.append(pltpu.SemaphoreType.DMA)
        for _ in range(kb):
            scratch.append(pltpu.SemaphoreType.DMA)
        scratch.append(pltpu.VMEM_SHARED((NP, f), jnp.float32))
        if stage_u:
            scratch.append(pltpu.VMEM_SHARED((NP, f), jnp.float32))

    @functools.partial(
        pl.kernel, out_type=out_type, mesh=mesh, scratch_types=scratch,
        compiler_params=pltpu.CompilerParams(use_tc_tiling_on_sc=False))
    def k(*refs):
        nu = len(fs)
        ng = nu * nt if with_gather else 0
        us = refs[:ng]
        gidx_hbm = refs[ng]
        sidx_hbm = refs[ng + 1]
        outs = refs[ng + 2:ng + 2 + nu]
        sc = refs[ng + 2 + nu:]
        gi_v, si_v = sc[0], sc[1]
        per_u = 2 * nbuf + kb + 1 + (1 if stage_u else 0)
        rows = []
        sems = []
        ssems = []
        accs = []
        stg = []
        for i in range(nu):
            grp = sc[2 + i * per_u:2 + (i + 1) * per_u]
            rows.append([grp[2 * k] for k in range(nbuf)])
            sems.append([grp[2 * k + 1] for k in range(nbuf)])
            ssems.append([grp[2 * nbuf + k] for k in range(kb)])
            accs.append(grp[2 * nbuf + kb])
            if stage_u:
                stg.append(grp[2 * nbuf + kb + 1])

        cid = lax.axis_index("c")
        sid = lax.axis_index("s")
        wid = sid * NC + cid

        def _fill_rows(val):
            for i, f in enumerate(fs):
                def frow(j, _, _r=rows[i][0], _f=f, _v=val):
                    for kk in range(_f // 16):
                        _r[j, pl.ds(16 * kk, 16)] = jnp.full(
                            (16,), _v, jnp.float32)
                    return 0
                lax.fori_loop(0, CH, frow, 0)

        def _start_gather(i, t, k, j):
            src = stg[i] if stage_u else us[i * nt + t]
            pltpu.async_copy(src.at[gi_v.at[j]], rows[i][k], sems[i][k])

        def _wait_gather(i, t, k):
            pltpu.make_async_copy(us[i * nt + t].at[pl.ds(0, CH)],
                                  rows[i][k], sems[i][k]).wait()

        def _start_scatter(i, b, k, j):
            pltpu.async_copy(rows[i][b], accs[i].at[si_v.at[j]],
                             ssems[i][k], add=True)

        def _wait_scatter(i, b, k):
            pltpu.make_async_copy(rows[i][b], accs[i].at[pl.ds(0, CH)],
                                  ssems[i][k]).wait()

        for t in range(nt):
            _fill_rows(0.0)
            for i in range(nu):
                for kk in range(STRIPE // CH):
                    pltpu.sync_copy(
                        rows[i][0],
                        accs[i].at[pl.ds(sid * STRIPE + kk * CH, CH)])
            if not with_gather:
                _fill_rows(1.0)
            if stage_u:
                for i in range(nu):
                    pltpu.sync_copy(
                        us[i * nt + t].at[pl.ds(sid * STRIPE, STRIPE)],
                        stg[i].at[pl.ds(sid * STRIPE, STRIPE)])
            plsc.subcore_barrier()
            pltpu.sync_copy(gidx_hbm.at[t, wid], gi_v)
            pltpu.sync_copy(sidx_hbm.at[t, wid], si_v)

            if with_gather:
                for k in range(kb):
                    for i in range(nu):
                        _start_gather(i, t, k, k)

                def group(g, _, _t=t):
                    for k in range(kb):
                        j = g * kb + k
                        for i in range(nu):
                            _wait_gather(i, _t, k)
                            _start_scatter(i, k, k, j)
                        pk = (k - 1) % kb
                        pj = j - 1 + kb
                        cond = (pj < NCH) if k >= 1 else (
                            (g >= 1) & (pj < NCH))

                        @pl.when(cond)
                        def _(_pk=pk, _pj=pj, _tt=_t):
                            for i in range(nu):
                                _wait_scatter(i, _pk, _pk)
                                _start_gather(i, _tt, _pk, _pj)
                    return 0
                lax.fori_loop(0, NCH // kb, group, 0)
                for k in range(kb):
                    for i in range(nu):
                        _wait_scatter(i, k, k)
            else:
                def group0(g, _):
                    for k in range(kb):
                        j = g * kb + k

                        @pl.when(g >= 1)
                        def _(_k=k):
                            for i in range(nu):
                                _wait_scatter(i, 0, _k)
                        for i in range(nu):
                            _start_scatter(i, 0, k, j)
                    return 0
                lax.fori_loop(0, NCH // kb, group0, 0)
                for k in range(kb):
                    for i in range(nu):
                        _wait_scatter(i, 0, k)
            plsc.subcore_barrier()
            for i in range(nu):
                pltpu.sync_copy(
                    accs[i].at[pl.ds(sid * STRIPE, STRIPE)],
                    outs[i].at[cid, t, pl.ds(sid * STRIPE, STRIPE)])
            plsc.subcore_barrier()

    return k


@functools.lru_cache(maxsize=None)
def _get_sc_pass(fs_key, nt, with_gather, kb=KB, stage_u=False):
    return _make_sc_pass(list(fs_key), nt, with_gather, kb, stage_u)


def _tc1_body(x_ref, degp_ref, w_ref, b_ref, encw_ref,
              phi_ref, ub_ref, dinv_ref):
    deg = degp_ref[0, 0] + degp_ref[1, 0] + 1.0
    dinv = lax.rsqrt(deg)
    dinv_ref[0] = dinv
    phi = jnp.maximum(
        jnp.dot(x_ref[0], w_ref[...],
                preferred_element_type=jnp.float32) + b_ref[...], 0.0)
    phi_ref[0] = phi
    ub_ref[0] = dinv[:, :1] * jnp.dot(phi, encw_ref[...],
                                      preferred_element_type=jnp.float32)


def _tc2_body(t, accb_ref, ub_ref, dinv_ref, wc_ref, uc_ref):
    d1 = dinv_ref[0][:, :1]
    enc = jnp.maximum(d1 * (accb_ref[0, 0] + accb_ref[1, 0] + ub_ref[0]), 0.0)
    uc_ref[...] = d1 * jnp.dot(enc, wc_ref[...],
                               preferred_element_type=jnp.float32)


def _tc3_body(t, accc_ref, uc_ref, dinv_ref, eps_ref, msk_ref, phi_ref,
              wz_ref, bz_ref, wda_ref, wdb_ref, pmu_ref, pstd_ref,
              mu_ref, zd_ref, udg_ref, kld_ref):
    i = pl.program_id(0)
    d1 = dinv_ref[0][:, :1]
    musd = d1 * (accc_ref[0, 0] + accc_ref[1, 0] + uc_ref[...])
    mu = musd[:, :ZD]
    std = _softplus(musd[:, ZD:])
    mu_ref[...] = mu
    z = mu + eps_ref[0] * std
    phiz = jnp.maximum(
        jnp.dot(z, wz_ref[...], preferred_element_type=jnp.float32)
        + bz_ref[...], 0.0)
    zd = msk_ref[0] * (2.0 * z)
    zd_ref[...] = zd
    udg_ref[...] = d1 * (
        jnp.dot(phi_ref[0], wda_ref[0], preferred_element_type=jnp.float32)
        + jnp.dot(phiz, wdb_ref[0], preferred_element_type=jnp.float32))
    pmu = pmu_ref[...]
    pstd = pstd_ref[...]
    term = (2.0 * (jnp.log(pstd + SEPS) - jnp.log(std + SEPS))
            + (std * std + (mu - pmu) ** 2) / (pstd * pstd + SEPS) - 1.0)
    row = lax.broadcasted_iota(jnp.int32, (NB, ZD), 0) + i * NB
    ksum = jnp.sum(jnp.where(row < N, term, 0.0))

    @pl.when(i == 0)
    def _():
        kld_ref[0, 0] = 0.0
    kld_ref[0, 0] += 0.5 * ksum / float(N)


def _tc4_body(t, accg_ref, accq_ref, udg_ref, dinv_ref, zd_ref, *rest):
    i = pl.program_id(0)
    d1 = dinv_ref[0][:, :1]
    g = d1 * (accg_ref[0, 0] + accg_ref[1, 0] + udg_ref[...])
    q = accq_ref[0, 0] + accq_ref[1, 0]
    sig = jax.nn.sigmoid(g[:, :HD])
    if t == 0:
        c1_ref, sle_ref = rest
        c1_ref[...] = sig * jnp.tanh(g[:, HD:])
    else:
        c1in_ref, h_ref, sle_ref = rest
        h_ref[...] = sig * jnp.tanh(c1in_ref[...])

    @pl.when(i == 0)
    def _():
        sle_ref[0, 0] = 0.0
    sle_ref[0, 0] += jnp.sum(q * zd_ref[...])


def _dec_body(zi_ref, zj_ref, s1_ref):
    i = pl.program_id(0)
    j = pl.program_id(1)

    @pl.when((i == 0) & (j == 0))
    def _():
        s1_ref[0, 0] = 0.0

    @pl.when(j >= i)
    def _():
        lg = lax.dot_general(zi_ref[...], zj_ref[...],
                             (((1,), (1,)), ((), ())),
                             preferred_element_type=jnp.float32)
        v = jnp.sum(_softplus_sum(lg))
        s1_ref[0, 0] += jnp.where(i == j, v, 2.0 * v)


def _np_spec(f, t):
    return pl.BlockSpec((1, NB, f), lambda i, _t=t: (_t, i, 0))


def _acc_spec(f, t):
    return pl.BlockSpec((2, 1, NB, f), lambda i, _t=t: (0, _t, i, 0))


def _flat_spec(f):
    return pl.BlockSpec((NB, f), lambda i: (i, 0))


def _full1(shape):
    nd = len(shape)
    return pl.BlockSpec(shape, lambda i, _s=nd: (0,) * _s)


_SMEM1 = pl.BlockSpec((1, 1), lambda *_: (0, 0), memory_space=pltpu.SMEM)
_F32 = jnp.float32


def kernel(x, edge_idx_list, params):
    p = params
    rkey = jax.random.key(42)
    eps = jnp.stack([
        jax.random.normal(jax.random.fold_in(rkey, 2 * t), (N, ZD),
                          _F32) for t in range(T)])
    msk = jnp.stack([
        jax.random.bernoulli(jax.random.fold_in(rkey, 2 * t + 1), 0.5,
                             (N, ZD)) for t in range(T)]).astype(_F32)
    eps = jnp.pad(eps, ((0, 0), (0, NP - N), (0, 0)))
    msk = jnp.pad(msk, ((0, 0), (0, NP - N), (0, 0)))
    xp = jnp.pad(x, ((0, 0), (0, NP - N), (0, 0)))

    pr = jax.nn.relu(p['prior_b'])
    pmu = (pr @ p['prior_mu_W'] + p['prior_mu_b']).reshape(1, ZD)
    pstd = jax.nn.softplus(pr @ p['prior_lv_W']
                           + p['prior_lv_b']).reshape(1, ZD)
    encw = p['enc_W'][:HD]
    wc = jnp.concatenate([p['enc_mu_W'], p['enc_lv_W']], axis=1)
    wd0 = jnp.concatenate([p['Wxi'], p['Wxc']], axis=1)
    wd1 = jnp.concatenate([p['Wxo'], p['Wxc']], axis=1)
    wda = jnp.stack([wd0[:HD], wd1[:HD]])
    wdb = jnp.stack([wd0[HD:], wd1[HD:]])
    bx = p['phi_x_b'].reshape(1, HD)
    bz = p['phi_z_b'].reshape(1, HD)

    ei = edge_idx_list
    row = ei[:, 0].reshape(T, NW, NCH, CH)
    col = ei[:, 1].reshape(T, NW, NCH, CH)

    (degp,) = _get_sc_pass((16,), T, False)(row, row)

    phi, ub, dinv = pl.pallas_call(
        _tc1_body,
        grid=(T, TB),
        in_specs=[pl.BlockSpec((1, NB, XD), lambda t, i: (t, i, 0)),
                  pl.BlockSpec((2, 1, NB, 16), lambda t, i: (0, t, i, 0)),
                  pl.BlockSpec((XD, HD), lambda t, i: (0, 0)),
                  pl.BlockSpec((1, HD), lambda t, i: (0, 0)),
                  pl.BlockSpec((HD, HD), lambda t, i: (0, 0))],
        out_specs=[pl.BlockSpec((1, NB, HD), lambda t, i: (t, i, 0)),
                   pl.BlockSpec((1, NB, HD), lambda t, i: (t, i, 0)),
                   pl.BlockSpec((1, NB, 16), lambda t, i: (t, i, 0))],
        out_shape=[jax.ShapeDtypeStruct((T, NP, HD), _F32),
                   jax.ShapeDtypeStruct((T, NP, HD), _F32),
                   jax.ShapeDtypeStruct((T, NP, 16), _F32)],
    )(xp, degp, p['phi_x_W'], bx, encw)

    sc_b = _get_sc_pass((HD,), 1, True, KB, True)
    sc_c = sc_b
    accb, uc, accc, mu_t, zd_t, udg_t, kld_t, s1_t = ([None] * T
        for _ in range(8))
    for t in range(T):
        (accb[t],) = sc_b(ub[t], row[t:t + 1], col[t:t + 1])
    for t in range(T):
        (uc[t],) = pl.pallas_call(
            functools.partial(_tc2_body, t),
            grid=(TB,),
            in_specs=[_acc_spec(HD, 0), _np_spec(HD, t), _np_spec(16, t),
                      _full1((HD, HD))],
            out_specs=[_flat_spec(HD)],
            out_shape=[jax.ShapeDtypeStruct((NP, HD), _F32)],
        )(accb[t], ub, dinv, wc)
        (accc[t],) = sc_c(uc[t], row[t:t + 1], col[t:t + 1])
    for t in range(T):
        mu_t[t], zd_t[t], udg_t[t], kld_t[t] = pl.pallas_call(
            functools.partial(_tc3_body, t),
            grid=(TB,),
            in_specs=[_acc_spec(HD, 0), _flat_spec(HD), _np_spec(16, t),
                      _np_spec(ZD, t), _np_spec(ZD, t), _np_spec(HD, t),
                      _full1((ZD, HD)), _full1((1, HD)),
                      pl.BlockSpec((1, HD, 2 * HD), lambda i, _t=t: (_t, 0, 0)),
                      pl.BlockSpec((1, HD, 2 * HD), lambda i, _t=t: (_t, 0, 0)),
                      _full1((1, ZD)), _full1((1, ZD))],
            out_specs=[_flat_spec(ZD), _flat_spec(ZD), _flat_spec(2 * HD),
                       _SMEM1],
            out_shape=[jax.ShapeDtypeStruct((NP, ZD), _F32),
                       jax.ShapeDtypeStruct((NP, ZD), _F32),
                       jax.ShapeDtypeStruct((NP, 2 * HD), _F32),
                       jax.ShapeDtypeStruct((1, 1), _F32)],
        )(accc[t], uc[t], dinv, eps, msk, phi, p['phi_z_W'], bz, wda, wdb,
          pmu, pstd)

    accg, accq = _get_sc_pass((2 * HD, ZD), T, True)(
        udg_t[0], udg_t[1], zd_t[0], zd_t[1], row, col)

    for t in range(T):
        (s1_t[t],) = pl.pallas_call(
            _dec_body,
            grid=(DB, DB),
            in_specs=[pl.BlockSpec((BM, ZD), lambda i, j: (i, 0)),
                      pl.BlockSpec((BM, ZD), lambda i, j: (j, 0))],
            out_specs=[pl.BlockSpec((1, 1), lambda i, j: (0, 0),
                                    memory_space=pltpu.SMEM)],
            out_shape=[jax.ShapeDtypeStruct((1, 1), _F32)],
        )(zd_t[t], zd_t[t])

    c1, sle0 = pl.pallas_call(
        functools.partial(_tc4_body, 0),
        grid=(TB,),
        in_specs=[_acc_spec(2 * HD, 0), _acc_spec(ZD, 0), _flat_spec(2 * HD),
                  _np_spec(16, 0), _flat_spec(ZD)],
        out_specs=[_flat_spec(HD), _SMEM1],
        out_shape=[jax.ShapeDtypeStruct((NP, HD), _F32),
                   jax.ShapeDtypeStruct((1, 1), _F32)],
    )(accg, accq, udg_t[0], dinv, zd_t[0])
    h_out, sle1 = pl.pallas_call(
        functools.partial(_tc4_body, 1),
        grid=(TB,),
        in_specs=[_acc_spec(2 * HD, 1), _acc_spec(ZD, 1), _flat_spec(2 * HD),
                  _np_spec(16, 1), _flat_spec(ZD), _flat_spec(HD)],
        out_specs=[_flat_spec(HD), _SMEM1],
        out_shape=[jax.ShapeDtypeStruct((NP, HD), _F32),
                   jax.ShapeDtypeStruct((1, 1), _F32)],
    )(accg, accq, udg_t[1], dinv, zd_t[1], c1)

    padc = float(NP * NP - N * N)
    sp0 = jnp.log(1.0 + jnp.exp(_F32(0.0)))
    nll = ((s1_t[0][0, 0] - padc * sp0 - sle0[0, 0])
           + (s1_t[1][0, 0] - padc * sp0 - sle1[0, 0])) / float(N * N)
    kld_s = kld_t[0][0, 0] + kld_t[1][0, 0]
    mus = jnp.stack([mu_t[0][:N], mu_t[1][:N]])
    h = h_out[None, :N, :]
    return kld_s, nll, mus, h


# speedup vs baseline: 1.1271x; 1.0515x over previous
import functools

import jax
import jax.numpy as jnp
from jax import lax
from jax.experimental import pallas as pl
from jax.experimental.pallas import tpu as pltpu
from jax.experimental.pallas import tpu_sc as plsc

XD = 128
HD = 32
ZD = 16
T = 2
N = 10000
E = 320000
NP = 10240
NB = 2048
TB = NP // NB
BM = 2048
DB = NP // BM
NC = 2
NS = 16
NW = NC * NS
EPW = E // NW
CH = 80
NCH = EPW // CH
KB = 5
STRIPE = NP // NS
SEPS = 1e-8


def _softplus(v):
    return jnp.maximum(v, 0.0) + jnp.log1p(jnp.exp(-jnp.abs(v)))


def _softplus_sum(v):
    return jnp.maximum(v, 0.0) + jnp.log(1.0 + jnp.exp(-jnp.abs(v)))


def _make_sc_pass(fs, nt, with_gather, kb=KB, stage_u=False):
    mesh = plsc.VectorSubcoreMesh(core_axis_name="c", subcore_axis_name="s",
                                  num_cores=NC, num_subcores=NS)
    out_type = tuple(
        jax.ShapeDtypeStruct((NC, nt, NP, f), jnp.float32) for f in fs)
    scratch = [
        pltpu.VMEM((NCH, CH), jnp.int32),
        pltpu.VMEM((NCH, CH), jnp.int32),
    ]
    nbuf = kb if with_gather else 1
    for f in fs:
        for _ in range(nbuf):
            scratch.append(pltpu.VMEM((CH, f), jnp.float32))
            scratch.append(pltpu.SemaphoreType.DMA)
        for _ in range(kb):
            scratch.append(pltpu.SemaphoreType.DMA)
        scratch.append(pltpu.VMEM_SHARED((NP, f), jnp.float32))
        if stage_u:
            scratch.append(pltpu.VMEM_SHARED((NP, f), jnp.float32))

    @functools.partial(
        pl.kernel, out_type=out_type, mesh=mesh, scratch_types=scratch,
        compiler_params=pltpu.CompilerParams(use_tc_tiling_on_sc=False))
    def k(*refs):
        nu = len(fs)
        ng = nu * nt if with_gather else 0
        us = refs[:ng]
        gidx_hbm = refs[ng]
        sidx_hbm = refs[ng + 1]
        outs = refs[ng + 2:ng + 2 + nu]
        sc = refs[ng + 2 + nu:]
        gi_v, si_v = sc[0], sc[1]
        per_u = 2 * nbuf + kb + 1 + (1 if stage_u else 0)
        rows = []
        sems = []
        ssems = []
        accs = []
        stg = []
        for i in range(nu):
            grp = sc[2 + i * per_u:2 + (i + 1) * per_u]
            rows.append([grp[2 * k] for k in range(nbuf)])
            sems.append([grp[2 * k + 1] for k in range(nbuf)])
            ssems.append([grp[2 * nbuf + k] for k in range(kb)])
            accs.append(grp[2 * nbuf + kb])
            if stage_u:
                stg.append(grp[2 * nbuf + kb + 1])

        cid = lax.axis_index("c")
        sid = lax.axis_index("s")
        wid = sid * NC + cid

        def _fill_rows(val):
            for i, f in enumerate(fs):
                def frow(j, _, _r=rows[i][0], _f=f, _v=val):
                    for kk in range(_f // 16):
                        _r[j, pl.ds(16 * kk, 16)] = jnp.full(
                            (16,), _v, jnp.float32)
                    return 0
                lax.fori_loop(0, CH, frow, 0)

        def _start_gather(i, t, k, j):
            src = stg[i] if stage_u else us[i * nt + t]
            pltpu.async_copy(src.at[gi_v.at[j]], rows[i][k], sems[i][k])

        def _wait_gather(i, t, k):
            pltpu.make_async_copy(us[i * nt + t].at[pl.ds(0, CH)],
                                  rows[i][k], sems[i][k]).wait()

        def _start_scatter(i, b, k, j):
            pltpu.async_copy(rows[i][b], accs[i].at[si_v.at[j]],
                             ssems[i][k], add=True)

        def _wait_scatter(i, b, k):
            pltpu.make_async_copy(rows[i][b], accs[i].at[pl.ds(0, CH)],
                                  ssems[i][k]).wait()

        for t in range(nt):
            _fill_rows(0.0)
            for i in range(nu):
                for kk in range(STRIPE // CH):
                    pltpu.sync_copy(
                        rows[i][0],
                        accs[i].at[pl.ds(sid * STRIPE + kk * CH, CH)])
            if not with_gather:
                _fill_rows(1.0)
            if stage_u:
                for i in range(nu):
                    pltpu.sync_copy(
                        us[i * nt + t].at[pl.ds(sid * STRIPE, STRIPE)],
                        stg[i].at[pl.ds(sid * STRIPE, STRIPE)])
            plsc.subcore_barrier()
            pltpu.sync_copy(gidx_hbm.at[t, wid], gi_v)
            pltpu.sync_copy(sidx_hbm.at[t, wid], si_v)

            if with_gather:
                for k in range(kb):
                    for i in range(nu):
                        _start_gather(i, t, k, k)

                def group(g, _, _t=t):
                    for k in range(kb):
                        j = g * kb + k
                        for i in range(nu):
                            _wait_gather(i, _t, k)
                            _start_scatter(i, k, k, j)
                        pk = (k - 1) % kb
                        pj = j - 1 + kb
                        cond = (pj < NCH) if k >= 1 else (
                            (g >= 1) & (pj < NCH))

                        @pl.when(cond)
                        def _(_pk=pk, _pj=pj, _tt=_t):
                            for i in range(nu):
                                _wait_scatter(i, _pk, _pk)
                                _start_gather(i, _tt, _pk, _pj)
                    return 0
                lax.fori_loop(0, NCH // kb, group, 0)
                for k in range(kb):
                    for i in range(nu):
                        _wait_scatter(i, k, k)
            else:
                def group0(g, _):
                    for k in range(kb):
                        j = g * kb + k

                        @pl.when(g >= 1)
                        def _(_k=k):
                            for i in range(nu):
                                _wait_scatter(i, 0, _k)
                        for i in range(nu):
                            _start_scatter(i, 0, k, j)
                    return 0
                lax.fori_loop(0, NCH // kb, group0, 0)
                for k in range(kb):
                    for i in range(nu):
                        _wait_scatter(i, 0, k)
            plsc.subcore_barrier()
            for i in range(nu):
                pltpu.sync_copy(
                    accs[i].at[pl.ds(sid * STRIPE, STRIPE)],
                    outs[i].at[cid, t, pl.ds(sid * STRIPE, STRIPE)])
            plsc.subcore_barrier()

    return k


@functools.lru_cache(maxsize=None)
def _get_sc_pass(fs_key, nt, with_gather, kb=KB, stage_u=False):
    return _make_sc_pass(list(fs_key), nt, with_gather, kb, stage_u)


def _tc1_body(x_ref, degp_ref, w_ref, b_ref, encw_ref,
              phi_ref, ub_ref, dinv_ref):
    deg = degp_ref[0, 0] + degp_ref[1, 0] + 1.0
    dinv = lax.rsqrt(deg)
    dinv_ref[0] = dinv
    phi = jnp.maximum(
        jnp.dot(x_ref[0], w_ref[...],
                preferred_element_type=jnp.float32) + b_ref[...], 0.0)
    phi_ref[0] = phi
    ub_ref[0] = dinv[:, :1] * jnp.dot(phi, encw_ref[...],
                                      preferred_element_type=jnp.float32)


def _tc2_body(t, accb_ref, ub_ref, dinv_ref, wc_ref, uc_ref):
    d1 = dinv_ref[0][:, :1]
    enc = jnp.maximum(d1 * (accb_ref[0, 0] + accb_ref[1, 0] + ub_ref[0]), 0.0)
    uc_ref[...] = d1 * jnp.dot(enc, wc_ref[...],
                               preferred_element_type=jnp.float32)


def _tc3_body(t, accc_ref, uc_ref, dinv_ref, eps_ref, msk_ref, phi_ref,
              wz_ref, bz_ref, wda_ref, wdb_ref, pmu_ref, pstd_ref,
              mu_ref, zd_ref, udg_ref, kld_ref):
    i = pl.program_id(0)
    d1 = dinv_ref[0][:, :1]
    musd = d1 * (accc_ref[0, 0] + accc_ref[1, 0] + uc_ref[...])
    mu = musd[:, :ZD]
    std = _softplus(musd[:, ZD:])
    mu_ref[...] = mu
    z = mu + eps_ref[0] * std
    phiz = jnp.maximum(
        jnp.dot(z, wz_ref[...], preferred_element_type=jnp.float32)
        + bz_ref[...], 0.0)
    zd = msk_ref[0] * (2.0 * z)
    zd_ref[...] = zd
    udg_ref[...] = d1 * (
        jnp.dot(phi_ref[0], wda_ref[0], preferred_element_type=jnp.float32)
        + jnp.dot(phiz, wdb_ref[0], preferred_element_type=jnp.float32))
    pmu = pmu_ref[...]
    pstd = pstd_ref[...]
    term = (2.0 * (jnp.log(pstd + SEPS) - jnp.log(std + SEPS))
            + (std * std + (mu - pmu) ** 2) / (pstd * pstd + SEPS) - 1.0)
    row = lax.broadcasted_iota(jnp.int32, (NB, ZD), 0) + i * NB
    ksum = jnp.sum(jnp.where(row < N, term, 0.0))

    @pl.when(i == 0)
    def _():
        kld_ref[0, 0] = 0.0
    kld_ref[0, 0] += 0.5 * ksum / float(N)


def _tc4_body(t, accg_ref, accq_ref, udg_ref, dinv_ref, zd_ref, *rest):
    i = pl.program_id(0)
    d1 = dinv_ref[0][:, :1]
    g = d1 * (accg_ref[0, 0] + accg_ref[1, 0] + udg_ref[...])
    q = accq_ref[0, 0] + accq_ref[1, 0]
    sig = jax.nn.sigmoid(g[:, :HD])
    if t == 0:
        c1_ref, sle_ref = rest
        c1_ref[...] = sig * jnp.tanh(g[:, HD:])
    else:
        c1in_ref, h_ref, sle_ref = rest
        h_ref[...] = sig * jnp.tanh(c1in_ref[...])

    @pl.when(i == 0)
    def _():
        sle_ref[0, 0] = 0.0
    sle_ref[0, 0] += jnp.sum(q * zd_ref[...])


def _dec_body(zi_ref, zj_ref, s1_ref):
    i = pl.program_id(0)
    j = pl.program_id(1)

    @pl.when((i == 0) & (j == 0))
    def _():
        s1_ref[0, 0] = 0.0

    @pl.when(j >= i)
    def _():
        lg = lax.dot_general(zi_ref[...], zj_ref[...],
                             (((1,), (1,)), ((), ())),
                             preferred_element_type=jnp.float32)
        v = jnp.sum(_softplus_sum(lg))
        s1_ref[0, 0] += jnp.where(i == j, v, 2.0 * v)


def _np_spec(f, t):
    return pl.BlockSpec((1, NB, f), lambda i, _t=t: (_t, i, 0))


def _acc_spec(f, t):
    return pl.BlockSpec((2, 1, NB, f), lambda i, _t=t: (0, _t, i, 0))


def _flat_spec(f):
    return pl.BlockSpec((NB, f), lambda i: (i, 0))


def _full1(shape):
    nd = len(shape)
    return pl.BlockSpec(shape, lambda i, _s=nd: (0,) * _s)


_SMEM1 = pl.BlockSpec((1, 1), lambda *_: (0, 0), memory_space=pltpu.SMEM)
_F32 = jnp.float32


@functools.lru_cache(maxsize=1)
def _rng_consts():
    import numpy as np
    with jax.ensure_compile_time_eval():
        rkey = jax.random.key(42)
        eps = np.stack([
            np.asarray(jax.random.normal(jax.random.fold_in(rkey, 2 * t),
                                         (N, ZD), _F32)) for t in range(T)])
        msk = np.stack([
            np.asarray(jax.random.bernoulli(
                jax.random.fold_in(rkey, 2 * t + 1), 0.5, (N, ZD)))
            for t in range(T)]).astype(np.float32)
    eps = np.pad(eps, ((0, 0), (0, NP - N), (0, 0)))
    msk = np.pad(msk, ((0, 0), (0, NP - N), (0, 0)))
    return eps, msk


def kernel(x, edge_idx_list, params):
    p = params
    eps, msk = _rng_consts()
    xp = jnp.pad(x, ((0, 0), (0, NP - N), (0, 0)))

    pr = jax.nn.relu(p['prior_b'])
    pmu = (pr @ p['prior_mu_W'] + p['prior_mu_b']).reshape(1, ZD)
    pstd = jax.nn.softplus(pr @ p['prior_lv_W']
                           + p['prior_lv_b']).reshape(1, ZD)
    encw = p['enc_W'][:HD]
    wc = jnp.concatenate([p['enc_mu_W'], p['enc_lv_W']], axis=1)
    wd0 = jnp.concatenate([p['Wxi'], p['Wxc']], axis=1)
    wd1 = jnp.concatenate([p['Wxo'], p['Wxc']], axis=1)
    wda = jnp.stack([wd0[:HD], wd1[:HD]])
    wdb = jnp.stack([wd0[HD:], wd1[HD:]])
    bx = p['phi_x_b'].reshape(1, HD)
    bz = p['phi_z_b'].reshape(1, HD)

    ei = edge_idx_list
    row = ei[:, 0].reshape(T, NW, NCH, CH)
    col = ei[:, 1].reshape(T, NW, NCH, CH)

    (degp,) = _get_sc_pass((16,), T, False)(row, row)

    phi, ub, dinv = pl.pallas_call(
        _tc1_body,
        grid=(T, TB),
        in_specs=[pl.BlockSpec((1, NB, XD), lambda t, i: (t, i, 0)),
                  pl.BlockSpec((2, 1, NB, 16), lambda t, i: (0, t, i, 0)),
                  pl.BlockSpec((XD, HD), lambda t, i: (0, 0)),
                  pl.BlockSpec((1, HD), lambda t, i: (0, 0)),
                  pl.BlockSpec((HD, HD), lambda t, i: (0, 0))],
        out_specs=[pl.BlockSpec((1, NB, HD), lambda t, i: (t, i, 0)),
                   pl.BlockSpec((1, NB, HD), lambda t, i: (t, i, 0)),
                   pl.BlockSpec((1, NB, 16), lambda t, i: (t, i, 0))],
        out_shape=[jax.ShapeDtypeStruct((T, NP, HD), _F32),
                   jax.ShapeDtypeStruct((T, NP, HD), _F32),
                   jax.ShapeDtypeStruct((T, NP, 16), _F32)],
    )(xp, degp, p['phi_x_W'], bx, encw)

    sc_b = _get_sc_pass((HD,), 1, True, KB, True)
    sc_c = sc_b
    accb, uc, accc, mu_t, zd_t, udg_t, kld_t, s1_t = ([None] * T
        for _ in range(8))
    for t in range(T):
        (accb[t],) = sc_b(ub[t], row[t:t + 1], col[t:t + 1])
    for t in range(T):
        (uc[t],) = pl.pallas_call(
            functools.partial(_tc2_body, t),
            grid=(TB,),
            in_specs=[_acc_spec(HD, 0), _np_spec(HD, t), _np_spec(16, t),
                      _full1((HD, HD))],
            out_specs=[_flat_spec(HD)],
            out_shape=[jax.ShapeDtypeStruct((NP, HD), _F32)],
        )(accb[t], ub, dinv, wc)
        (accc[t],) = sc_c(uc[t], row[t:t + 1], col[t:t + 1])
    for t in range(T):
        mu_t[t], zd_t[t], udg_t[t], kld_t[t] = pl.pallas_call(
            functools.partial(_tc3_body, t),
            grid=(TB,),
            in_specs=[_acc_spec(HD, 0), _flat_spec(HD), _np_spec(16, t),
                      _np_spec(ZD, t), _np_spec(ZD, t), _np_spec(HD, t),
                      _full1((ZD, HD)), _full1((1, HD)),
                      pl.BlockSpec((1, HD, 2 * HD), lambda i, _t=t: (_t, 0, 0)),
                      pl.BlockSpec((1, HD, 2 * HD), lambda i, _t=t: (_t, 0, 0)),
                      _full1((1, ZD)), _full1((1, ZD))],
            out_specs=[_flat_spec(ZD), _flat_spec(ZD), _flat_spec(2 * HD),
                       _SMEM1],
            out_shape=[jax.ShapeDtypeStruct((NP, ZD), _F32),
                       jax.ShapeDtypeStruct((NP, ZD), _F32),
                       jax.ShapeDtypeStruct((NP, 2 * HD), _F32),
                       jax.ShapeDtypeStruct((1, 1), _F32)],
        )(accc[t], uc[t], dinv, eps, msk, phi, p['phi_z_W'], bz, wda, wdb,
          pmu, pstd)

    accg, accq = _get_sc_pass((2 * HD, ZD), T, True)(
        udg_t[0], udg_t[1], zd_t[0], zd_t[1], row, col)

    for t in range(T):
        (s1_t[t],) = pl.pallas_call(
            _dec_body,
            grid=(DB, DB),
            in_specs=[pl.BlockSpec((BM, ZD), lambda i, j: (i, 0)),
                      pl.BlockSpec((BM, ZD), lambda i, j: (j, 0))],
            out_specs=[pl.BlockSpec((1, 1), lambda i, j: (0, 0),
                                    memory_space=pltpu.SMEM)],
            out_shape=[jax.ShapeDtypeStruct((1, 1), _F32)],
        )(zd_t[t], zd_t[t])

    c1, sle0 = pl.pallas_call(
        functools.partial(_tc4_body, 0),
        grid=(TB,),
        in_specs=[_acc_spec(2 * HD, 0), _acc_spec(ZD, 0), _flat_spec(2 * HD),
                  _np_spec(16, 0), _flat_spec(ZD)],
        out_specs=[_flat_spec(HD), _SMEM1],
        out_shape=[jax.ShapeDtypeStruct((NP, HD), _F32),
                   jax.ShapeDtypeStruct((1, 1), _F32)],
    )(accg, accq, udg_t[0], dinv, zd_t[0])
    h_out, sle1 = pl.pallas_call(
        functools.partial(_tc4_body, 1),
        grid=(TB,),
        in_specs=[_acc_spec(2 * HD, 1), _acc_spec(ZD, 1), _flat_spec(2 * HD),
                  _np_spec(16, 1), _flat_spec(ZD), _flat_spec(HD)],
        out_specs=[_flat_spec(HD), _SMEM1],
        out_shape=[jax.ShapeDtypeStruct((NP, HD), _F32),
                   jax.ShapeDtypeStruct((1, 1), _F32)],
    )(accg, accq, udg_t[1], dinv, zd_t[1], c1)

    padc = float(NP * NP - N * N)
    sp0 = jnp.log(1.0 + jnp.exp(_F32(0.0)))
    nll = ((s1_t[0][0, 0] - padc * sp0 - sle0[0, 0])
           + (s1_t[1][0, 0] - padc * sp0 - sle1[0, 0])) / float(N * N)
    kld_s = kld_t[0][0, 0] + kld_t[1][0, 0]
    mus = jnp.stack([mu_t[0][:N], mu_t[1][:N]])
    h = h_out[None, :N, :]
    return kld_s, nll, mus, h
